# Initial kernel scaffold; baseline (speedup 1.0000x reference)
import jax
import jax.numpy as jnp
from jax.experimental import pallas as pl


def kernel(x, edge_index, edge_attr, params):
    raise NotImplementedError("write your pallas kernel here")


# speedup vs baseline: 4.7937x; 4.7937x over previous
import functools

import jax
import jax.numpy as jnp
from jax import lax
from jax.experimental import pallas as pl
from jax.experimental.pallas import tpu as pltpu
from jax.experimental.pallas import tpu_sc as plsc

N = 10000
E = 160000
H = 128
NPAD = 10240
EP = 163840
NTILES = 32
PER_TILE = EP // NTILES
CH = 128
NCH = PER_TILE // CH
STRIPE = NPAD // 16
RBLK = 2048
NHB = NPAD // RBLK

f32 = jnp.float32


@functools.cache
def _mesh():
    return plsc.VectorSubcoreMesh(core_axis_name="c", subcore_axis_name="s")


def _Z():
    return jnp.int32(0)


def _dot(a, b):
    return lax.dot_general(a, b, (((1,), (0,)), ((), ())),
                           preferred_element_type=f32)


def _silu(h):
    return h * (1.0 / (1.0 + jnp.exp(-h)))


def _mlp(xp, p, ln):
    n, din = xp.shape
    ws = [w.astype(f32) for w in p["Ws"]]
    dout = ws[3].shape[1]
    if dout != H:
        ws = ws[:3] + [jnp.pad(ws[3], ((0, 0), (0, H - dout)))]
    bs = jnp.stack([jnp.pad(b, (0, H - b.shape[0])) for b in p["bs"]])
    gb = (jnp.stack([p["ln_g"], p["ln_b"]]) if ln
          else jnp.zeros((2, H), f32))

    def body(x_ref, w0, w1, w2, w3, bs_ref, gb_ref, o_ref):
        h = x_ref[...]
        b = bs_ref[...]
        for i, w in enumerate((w0, w1, w2)):
            h = _silu(_dot(h, w[...]) + b[i])
        h = _dot(h, w3[...]) + b[3]
        if ln:
            g = gb_ref[...]
            mu = jnp.mean(h, axis=-1, keepdims=True)
            var = jnp.mean((h - mu) ** 2, axis=-1, keepdims=True)
            h = (h - mu) * lax.rsqrt(var + 1e-5) * g[0] + g[1]
        o_ref[...] = h

    wspec = [pl.BlockSpec((ws[i].shape[0], H), lambda i: (_Z(), _Z()))
             for i in range(4)]
    return pl.pallas_call(
        body, grid=(n // RBLK,),
        in_specs=[pl.BlockSpec((RBLK, din), lambda i: (i, _Z()))] + wspec + [
            pl.BlockSpec((4, H), lambda i: (_Z(), _Z())),
            pl.BlockSpec((2, H), lambda i: (_Z(), _Z()))],
        out_specs=pl.BlockSpec((RBLK, H), lambda i: (i, _Z())),
        out_shape=jax.ShapeDtypeStruct((n, H), f32),
    )(xp, *ws, bs, gb)


def _deg_post(cnt):
    def body(c_ref, dv_ref, a1_ref):
        cb = c_ref[...]
        deg = (cb[0] + cb[1])[:, 0:1] + 1.0
        row = (lax.broadcasted_iota(jnp.int32, (RBLK, 1), 0)
               + pl.program_id(0) * RBLK)
        mask = (row < N).astype(f32)
        dv = mask * lax.rsqrt(deg)
        a1 = mask * (1.0 / deg - 1.0)
        dv_ref[...] = jnp.broadcast_to(dv, (RBLK, H))
        a1_ref[...] = jnp.broadcast_to(a1, (RBLK, H))

    return pl.pallas_call(
        body, grid=(NHB,),
        in_specs=[pl.BlockSpec((2, RBLK, 16), lambda i: (_Z(), i, _Z()))],
        out_specs=[pl.BlockSpec((RBLK, H), lambda i: (i, _Z()))] * 2,
        out_shape=[jax.ShapeDtypeStruct((NPAD, H), f32)] * 2,
    )(cnt)


def _ab(h_n, w1, w2):
    def body(x_ref, w1r, w2r, a_ref, b_ref):
        xx = x_ref[...]
        a_ref[...] = _dot(xx, w1r[...])
        b_ref[...] = _dot(xx, w2r[...])

    return pl.pallas_call(
        body, grid=(NHB,),
        in_specs=[pl.BlockSpec((RBLK, H), lambda i: (i, _Z())),
                  pl.BlockSpec((H, H), lambda i: (_Z(), _Z())),
                  pl.BlockSpec((H, H), lambda i: (_Z(), _Z()))],
        out_specs=[pl.BlockSpec((RBLK, H), lambda i: (i, _Z()))] * 2,
        out_shape=[jax.ShapeDtypeStruct((NPAD, H), f32)] * 2,
    )(h_n, w1, w2)


def _head_pre(G, h_e, w3, dv):
    def body(g_ref, he_ref, w3r, dv_ref, xw_ref, tp_ref):
        xw = g_ref[...] + _dot(he_ref[...], w3r[...])
        xw_ref[...] = xw
        tp_ref[...] = xw * dv_ref[...]

    return pl.pallas_call(
        body, grid=(NHB,),
        in_specs=[pl.BlockSpec((RBLK, H), lambda i: (i, _Z())),
                  pl.BlockSpec((RBLK, H), lambda i: (i, _Z())),
                  pl.BlockSpec((H, H), lambda i: (_Z(), _Z())),
                  pl.BlockSpec((RBLK, H), lambda i: (i, _Z()))],
        out_specs=[pl.BlockSpec((RBLK, H), lambda i: (i, _Z()))] * 2,
        out_shape=[jax.ShapeDtypeStruct((NPAD, H), f32)] * 2,
    )(G, h_e, w3, dv)


def _fix(xw_h, a1, dv, m):
    def body(xw_ref, a1_ref, dv_ref, m_ref, o_ref):
        mm = m_ref[...]
        o_ref[...] = (xw_ref[...] * a1_ref[...]
                      + dv_ref[...] * (mm[0] + mm[1]))

    return pl.pallas_call(
        body, grid=(NHB,),
        in_specs=[pl.BlockSpec((RBLK, H), lambda i: (i, _Z())),
                  pl.BlockSpec((RBLK, H), lambda i: (i, _Z())),
                  pl.BlockSpec((RBLK, H), lambda i: (i, _Z())),
                  pl.BlockSpec((2, RBLK, H), lambda i: (_Z(), i, _Z()))],
        out_specs=pl.BlockSpec((RBLK, H), lambda i: (i, _Z())),
        out_shape=jax.ShapeDtypeStruct((NPAD, H), f32),
    )(xw_h, a1, dv, m)


def _edge_big(G, h_e, fix, w3, bc, wo, bo):
    def body(g_ref, he_ref, fix_ref, w3r, bcr, wor, bor, ne_ref, hn_ref):
        he = he_ref[...]
        xw = g_ref[...] + _dot(he, w3r[...])
        flag = (pl.program_id(0) < NHB).astype(f32)
        pre = xw + bcr[...] + fix_ref[...] * flag
        ne = _dot(jnp.maximum(pre, 0.0), wor[...]) + bor[...]
        ne_ref[...] = ne
        hn_ref[...] = he + ne

    return pl.pallas_call(
        body, grid=(EP // RBLK,),
        in_specs=[pl.BlockSpec((RBLK, H), lambda i: (i, _Z())),
                  pl.BlockSpec((RBLK, H), lambda i: (i, _Z())),
                  pl.BlockSpec((RBLK, H),
                               lambda i: (jnp.minimum(
                                   i, jnp.int32(NHB - 1)), _Z())),
                  pl.BlockSpec((H, H), lambda i: (_Z(), _Z())),
                  pl.BlockSpec((1, H), lambda i: (_Z(), _Z())),
                  pl.BlockSpec((H, H), lambda i: (_Z(), _Z())),
                  pl.BlockSpec((1, H), lambda i: (_Z(), _Z()))],
        out_specs=[pl.BlockSpec((RBLK, H), lambda i: (i, _Z()))] * 2,
        out_shape=[jax.ShapeDtypeStruct((EP, H), f32)] * 2,
    )(G, h_e, fix, w3, bc.reshape(1, H), wo, bo.reshape(1, H))


def _node_pre(h_n, agg, w1, w2, dv):
    def body(hn_ref, ag_ref, w1r, w2r, dv_ref, xw_ref, tp_ref):
        ag = ag_ref[...]
        xw = _dot(hn_ref[...], w1r[...]) + _dot(ag[0] + ag[1], w2r[...])
        xw_ref[...] = xw
        tp_ref[...] = xw * dv_ref[...]

    return pl.pallas_call(
        body, grid=(NHB,),
        in_specs=[pl.BlockSpec((RBLK, H), lambda i: (i, _Z())),
                  pl.BlockSpec((2, RBLK, H), lambda i: (_Z(), i, _Z())),
                  pl.BlockSpec((H, H), lambda i: (_Z(), _Z())),
                  pl.BlockSpec((H, H), lambda i: (_Z(), _Z())),
                  pl.BlockSpec((RBLK, H), lambda i: (i, _Z()))],
        out_specs=[pl.BlockSpec((RBLK, H), lambda i: (i, _Z()))] * 2,
        out_shape=[jax.ShapeDtypeStruct((NPAD, H), f32)] * 2,
    )(h_n, agg, w1, w2, dv)


def _node_post(h_n, xw_n, m, a1, dv, bn, won, bon):
    def body(hn_ref, xw_ref, m_ref, a1_ref, dv_ref, bnr, wor, bor, o_ref):
        mm = m_ref[...]
        pre = (xw_ref[...] * (a1_ref[...] + 1.0)
               + dv_ref[...] * (mm[0] + mm[1]) + bnr[...])
        nn = _dot(jnp.maximum(pre, 0.0), wor[...]) + bor[...]
        o_ref[...] = hn_ref[...] + nn

    return pl.pallas_call(
        body, grid=(NHB,),
        in_specs=[pl.BlockSpec((RBLK, H), lambda i: (i, _Z())),
                  pl.BlockSpec((RBLK, H), lambda i: (i, _Z())),
                  pl.BlockSpec((2, RBLK, H), lambda i: (_Z(), i, _Z())),
                  pl.BlockSpec((RBLK, H), lambda i: (i, _Z())),
                  pl.BlockSpec((RBLK, H), lambda i: (i, _Z())),
                  pl.BlockSpec((1, H), lambda i: (_Z(), _Z())),
                  pl.BlockSpec((H, H), lambda i: (_Z(), _Z())),
                  pl.BlockSpec((1, H), lambda i: (_Z(), _Z()))],
        out_specs=pl.BlockSpec((RBLK, H), lambda i: (i, _Z())),
        out_shape=jax.ShapeDtypeStruct((NPAD, H), f32),
    )(h_n, xw_n, m, a1, dv, bn.reshape(1, H), won, bon.reshape(1, H))


def _sc_deg(dst_s, ones16, z16):
    @functools.partial(
        pl.kernel,
        out_type=jax.ShapeDtypeStruct((2 * NPAD, 16), f32),
        mesh=_mesh(),
        scratch_types=[pltpu.VMEM((CH,), jnp.int32),
                       pltpu.VMEM((CH, 16), f32),
                       pltpu.VMEM_SHARED((NPAD, 16), f32)],
    )
    def k(d_hbm, o_hbm, z_hbm, out_hbm, idx, ones_v, acc):
        c = lax.axis_index("c")
        s = lax.axis_index("s")
        pltpu.sync_copy(z_hbm, acc.at[pl.ds(s * jnp.int32(STRIPE), STRIPE)])
        pltpu.sync_copy(o_hbm, ones_v)
        plsc.subcore_barrier()
        base0 = (c * jnp.int32(16) + s) * jnp.int32(PER_TILE)

        @pl.loop(jnp.int32(0), jnp.int32(NCH))
        def _(ch):
            pltpu.sync_copy(d_hbm.at[pl.ds(base0 + ch * jnp.int32(CH), CH)], idx)
            pltpu.sync_copy(ones_v, acc.at[idx], add=True)

        plsc.subcore_barrier()
        pltpu.sync_copy(
            acc.at[pl.ds(s * jnp.int32(STRIPE), STRIPE)],
            out_hbm.at[pl.ds(c * jnp.int32(NPAD) + s * jnp.int32(STRIPE),
                             STRIPE)])

    return k(dst_s, ones16, z16).reshape(2, NPAD, 16)


def _sc_gather2(A, B, src_p, dst_g):
    @functools.partial(
        pl.kernel,
        out_type=jax.ShapeDtypeStruct((EP, H), f32),
        mesh=_mesh(),
        scratch_types=[pltpu.VMEM((CH,), jnp.int32),
                       pltpu.VMEM((CH,), jnp.int32),
                       pltpu.VMEM((CH, H), f32),
                       pltpu.VMEM((CH, H), f32)],
    )
    def k(a_hbm, b_hbm, s_hbm, d_hbm, g_hbm, i1, i2, bufa, bufb):
        c = lax.axis_index("c")
        s = lax.axis_index("s")
        base0 = (c * jnp.int32(16) + s) * jnp.int32(PER_TILE)

        @pl.loop(jnp.int32(0), jnp.int32(NCH))
        def _(ch):
            off = base0 + ch * jnp.int32(CH)
            pltpu.sync_copy(s_hbm.at[pl.ds(off, CH)], i1)
            pltpu.sync_copy(d_hbm.at[pl.ds(off, CH)], i2)
            pltpu.sync_copy(a_hbm.at[i1], bufa)
            pltpu.sync_copy(b_hbm.at[i2], bufb)

            @pl.loop(jnp.int32(0), jnp.int32(CH))
            def _(r):
                @pl.loop(jnp.int32(0), jnp.int32(H), step=jnp.int32(16))
                def _(kk):
                    sl = pl.ds(kk, 16)
                    bufa[r, sl] = bufa[r, sl] + bufb[r, sl]

            pltpu.sync_copy(bufa, g_hbm.at[pl.ds(off, CH)])

    return k(A, B, src_p, dst_g)


def _sc_msg(tab, src_p, dst_s, z128):
    @functools.partial(
        pl.kernel,
        out_type=jax.ShapeDtypeStruct((2 * NPAD, H), f32),
        mesh=_mesh(),
        scratch_types=[pltpu.VMEM((CH,), jnp.int32),
                       pltpu.VMEM((CH,), jnp.int32),
                       pltpu.VMEM((CH, H), f32),
                       pltpu.VMEM_SHARED((NPAD, H), f32)],
    )
    def k(t_hbm, s_hbm, d_hbm, z_hbm, out_hbm, i1, i2, buf, acc):
        c = lax.axis_index("c")
        s = lax.axis_index("s")
        pltpu.sync_copy(z_hbm, acc.at[pl.ds(s * jnp.int32(STRIPE), STRIPE)])
        plsc.subcore_barrier()
        base0 = (c * jnp.int32(16) + s) * jnp.int32(PER_TILE)

        @pl.loop(jnp.int32(0), jnp.int32(NCH))
        def _(ch):
            off = base0 + ch * jnp.int32(CH)
            pltpu.sync_copy(s_hbm.at[pl.ds(off, CH)], i1)
            pltpu.sync_copy(d_hbm.at[pl.ds(off, CH)], i2)
            pltpu.sync_copy(t_hbm.at[i1], buf)
            pltpu.sync_copy(buf, acc.at[i2], add=True)

        plsc.subcore_barrier()
        pltpu.sync_copy(
            acc.at[pl.ds(s * jnp.int32(STRIPE), STRIPE)],
            out_hbm.at[pl.ds(c * jnp.int32(NPAD) + s * jnp.int32(STRIPE),
                             STRIPE)])

    return k(tab, src_p, dst_s, z128).reshape(2, NPAD, H)


def _sc_agg(V, dst_s, z128):
    @functools.partial(
        pl.kernel,
        out_type=jax.ShapeDtypeStruct((2 * NPAD, H), f32),
        mesh=_mesh(),
        scratch_types=[pltpu.VMEM((CH,), jnp.int32),
                       pltpu.VMEM((CH, H), f32),
                       pltpu.VMEM_SHARED((NPAD, H), f32)],
    )
    def k(v_hbm, d_hbm, z_hbm, out_hbm, i2, buf, acc):
        c = lax.axis_index("c")
        s = lax.axis_index("s")
        pltpu.sync_copy(z_hbm, acc.at[pl.ds(s * jnp.int32(STRIPE), STRIPE)])
        plsc.subcore_barrier()
        base0 = (c * jnp.int32(16) + s) * jnp.int32(PER_TILE)

        @pl.loop(jnp.int32(0), jnp.int32(NCH))
        def _(ch):
            off = base0 + ch * jnp.int32(CH)
            pltpu.sync_copy(d_hbm.at[pl.ds(off, CH)], i2)
            pltpu.sync_copy(v_hbm.at[pl.ds(off, CH)], buf)
            pltpu.sync_copy(buf, acc.at[i2], add=True)

        plsc.subcore_barrier()
        pltpu.sync_copy(
            acc.at[pl.ds(s * jnp.int32(STRIPE), STRIPE)],
            out_hbm.at[pl.ds(c * jnp.int32(NPAD) + s * jnp.int32(STRIPE),
                             STRIPE)])

    return k(V, dst_s, z128).reshape(2, NPAD, H)


def kernel(x, edge_index, edge_attr, params):
    src = edge_index[0].astype(jnp.int32)
    dst = edge_index[1].astype(jnp.int32)
    src_p = jnp.pad(src, (0, EP - E))
    dst_g = jnp.pad(dst, (0, EP - E))
    dst_s = jnp.pad(dst, (0, EP - E), constant_values=N)
    x_p = jnp.pad(x.astype(f32), ((0, NPAD - N), (0, 0)))
    ea_p = jnp.pad(edge_attr.astype(f32), ((0, EP - E), (0, 0)))
    z128 = jnp.zeros((STRIPE, H), f32)
    z16 = jnp.zeros((STRIPE, 16), f32)
    ones16 = jnp.ones((CH, 16), f32)

    h_n = _mlp(x_p, params["enc_node"], ln=True)
    h_e = _mlp(ea_p, params["enc_edge"], ln=True)
    cnt = _sc_deg(dst_s, ones16, z16)
    dv, a1 = _deg_post(cnt)

    for blk in params["blocks"]:
        eb, nb = blk["eb"], blk["nb"]
        w1, w2, w3 = eb["Wc"][:H], eb["Wc"][H:2 * H], eb["Wc"][2 * H:]
        A, B = _ab(h_n, w1, w2)
        G = _sc_gather2(A, B, src_p, dst_g)
        xw_h, tab = _head_pre(G, h_e, w3, dv)
        m = _sc_msg(tab, src_p, dst_s, z128)
        fx = _fix(xw_h, a1, dv, m)
        new_e, h_e = _edge_big(G, h_e, fx, w3, eb["bc"], eb["Wo"], eb["bo"])
        agg = _sc_agg(new_e, dst_s, z128)
        wn1, wn2 = nb["Wc"][:H], nb["Wc"][H:]
        xw_n, tab_n = _node_pre(h_n, agg, wn1, wn2, dv)
        mn = _sc_msg(tab_n, src_p, dst_s, z128)
        h_n = _node_post(h_n, xw_n, mn, a1, dv, nb["bc"], nb["Wo"], nb["bo"])

    out = _mlp(h_n, params["dec"], ln=False)
    return out[:N, :2]


# speedup vs baseline: 5.9460x; 1.2404x over previous
import functools

import jax
import jax.numpy as jnp
from jax import lax
from jax.experimental import pallas as pl
from jax.experimental.pallas import tpu as pltpu
from jax.experimental.pallas import tpu_sc as plsc

N = 10000
E = 160000
H = 128
NPAD = 10240
EP = 163840
NTILES = 32
PER_TILE = EP // NTILES
CH = 128
NCH = PER_TILE // CH
STRIPE = NPAD // 16
RBLK = 2048
NHB = NPAD // RBLK

f32 = jnp.float32


@functools.cache
def _mesh():
    return plsc.VectorSubcoreMesh(core_axis_name="c", subcore_axis_name="s")


def _Z():
    return jnp.int32(0)


def _dot(a, b):
    return lax.dot_general(a, b, (((1,), (0,)), ((), ())),
                           preferred_element_type=f32)


def _silu(h):
    return h * (1.0 / (1.0 + jnp.exp(-h)))


def _mlp(xp, p, ln):
    n, din = xp.shape
    ws = [w.astype(f32) for w in p["Ws"]]
    dout = ws[3].shape[1]
    if dout != H:
        ws = ws[:3] + [jnp.pad(ws[3], ((0, 0), (0, H - dout)))]
    bs = jnp.stack([jnp.pad(b, (0, H - b.shape[0])) for b in p["bs"]])
    gb = (jnp.stack([p["ln_g"], p["ln_b"]]) if ln
          else jnp.zeros((2, H), f32))

    def body(x_ref, w0, w1, w2, w3, bs_ref, gb_ref, o_ref):
        h = x_ref[...]
        b = bs_ref[...]
        for i, w in enumerate((w0, w1, w2)):
            h = _silu(_dot(h, w[...]) + b[i])
        h = _dot(h, w3[...]) + b[3]
        if ln:
            g = gb_ref[...]
            mu = jnp.mean(h, axis=-1, keepdims=True)
            var = jnp.mean((h - mu) ** 2, axis=-1, keepdims=True)
            h = (h - mu) * lax.rsqrt(var + 1e-5) * g[0] + g[1]
        o_ref[...] = h

    wspec = [pl.BlockSpec((ws[i].shape[0], H), lambda i: (_Z(), _Z()))
             for i in range(4)]
    return pl.pallas_call(
        body, grid=(n // RBLK,),
        in_specs=[pl.BlockSpec((RBLK, din), lambda i: (i, _Z()))] + wspec + [
            pl.BlockSpec((4, H), lambda i: (_Z(), _Z())),
            pl.BlockSpec((2, H), lambda i: (_Z(), _Z()))],
        out_specs=pl.BlockSpec((RBLK, H), lambda i: (i, _Z())),
        out_shape=jax.ShapeDtypeStruct((n, H), f32),
    )(xp, *ws, bs, gb)


def _deg_post(cnt):
    def body(c_ref, dv_ref, a1_ref):
        cb = c_ref[...]
        deg = (cb[0] + cb[1])[:, 0:1] + 1.0
        row = (lax.broadcasted_iota(jnp.int32, (RBLK, 1), 0)
               + pl.program_id(0) * RBLK)
        mask = (row < N).astype(f32)
        dv = mask * lax.rsqrt(deg)
        a1 = mask * (1.0 / deg - 1.0)
        dv_ref[...] = jnp.broadcast_to(dv, (RBLK, H))
        a1_ref[...] = jnp.broadcast_to(a1, (RBLK, H))

    return pl.pallas_call(
        body, grid=(NHB,),
        in_specs=[pl.BlockSpec((2, RBLK, 16), lambda i: (_Z(), i, _Z()))],
        out_specs=[pl.BlockSpec((RBLK, H), lambda i: (i, _Z()))] * 2,
        out_shape=[jax.ShapeDtypeStruct((NPAD, H), f32)] * 2,
    )(cnt)


def _ab(h_n, w1, w2):
    def body(x_ref, w1r, w2r, a_ref, b_ref):
        xx = x_ref[...]
        a_ref[...] = _dot(xx, w1r[...])
        b_ref[...] = _dot(xx, w2r[...])

    return pl.pallas_call(
        body, grid=(NHB,),
        in_specs=[pl.BlockSpec((RBLK, H), lambda i: (i, _Z())),
                  pl.BlockSpec((H, H), lambda i: (_Z(), _Z())),
                  pl.BlockSpec((H, H), lambda i: (_Z(), _Z()))],
        out_specs=[pl.BlockSpec((RBLK, H), lambda i: (i, _Z()))] * 2,
        out_shape=[jax.ShapeDtypeStruct((NPAD, H), f32)] * 2,
    )(h_n, w1, w2)


def _head_pre(G, h_e, w3, dv):
    def body(g_ref, he_ref, w3r, dv_ref, xw_ref, tp_ref):
        xw = g_ref[...] + _dot(he_ref[...], w3r[...])
        xw_ref[...] = xw
        tp_ref[...] = xw * dv_ref[...]

    return pl.pallas_call(
        body, grid=(NHB,),
        in_specs=[pl.BlockSpec((RBLK, H), lambda i: (i, _Z())),
                  pl.BlockSpec((RBLK, H), lambda i: (i, _Z())),
                  pl.BlockSpec((H, H), lambda i: (_Z(), _Z())),
                  pl.BlockSpec((RBLK, H), lambda i: (i, _Z()))],
        out_specs=[pl.BlockSpec((RBLK, H), lambda i: (i, _Z()))] * 2,
        out_shape=[jax.ShapeDtypeStruct((NPAD, H), f32)] * 2,
    )(G, h_e, w3, dv)


def _fix(xw_h, a1, dv, m):
    def body(xw_ref, a1_ref, dv_ref, m_ref, o_ref):
        mm = m_ref[...]
        o_ref[...] = (xw_ref[...] * a1_ref[...]
                      + dv_ref[...] * (mm[0] + mm[1]))

    return pl.pallas_call(
        body, grid=(NHB,),
        in_specs=[pl.BlockSpec((RBLK, H), lambda i: (i, _Z())),
                  pl.BlockSpec((RBLK, H), lambda i: (i, _Z())),
                  pl.BlockSpec((RBLK, H), lambda i: (i, _Z())),
                  pl.BlockSpec((2, RBLK, H), lambda i: (_Z(), i, _Z()))],
        out_specs=pl.BlockSpec((RBLK, H), lambda i: (i, _Z())),
        out_shape=jax.ShapeDtypeStruct((NPAD, H), f32),
    )(xw_h, a1, dv, m)


def _edge_big(G, h_e, fix, w3, bc, wo, bo):
    def body(g_ref, he_ref, fix_ref, w3r, bcr, wor, bor, ne_ref, hn_ref):
        he = he_ref[...]
        xw = g_ref[...] + _dot(he, w3r[...])
        flag = (pl.program_id(0) < NHB).astype(f32)
        pre = xw + bcr[...] + fix_ref[...] * flag
        ne = _dot(jnp.maximum(pre, 0.0), wor[...]) + bor[...]
        ne_ref[...] = ne
        hn_ref[...] = he + ne

    return pl.pallas_call(
        body, grid=(EP // RBLK,),
        in_specs=[pl.BlockSpec((RBLK, H), lambda i: (i, _Z())),
                  pl.BlockSpec((RBLK, H), lambda i: (i, _Z())),
                  pl.BlockSpec((RBLK, H),
                               lambda i: (jnp.minimum(
                                   i, jnp.int32(NHB - 1)), _Z())),
                  pl.BlockSpec((H, H), lambda i: (_Z(), _Z())),
                  pl.BlockSpec((1, H), lambda i: (_Z(), _Z())),
                  pl.BlockSpec((H, H), lambda i: (_Z(), _Z())),
                  pl.BlockSpec((1, H), lambda i: (_Z(), _Z()))],
        out_specs=[pl.BlockSpec((RBLK, H), lambda i: (i, _Z()))] * 2,
        out_shape=[jax.ShapeDtypeStruct((EP, H), f32)] * 2,
    )(G, h_e, fix, w3, bc.reshape(1, H), wo, bo.reshape(1, H))


def _node_pre(h_n, agg, w1, w2, dv):
    def body(hn_ref, ag_ref, w1r, w2r, dv_ref, xw_ref, tp_ref):
        ag = ag_ref[...]
        xw = _dot(hn_ref[...], w1r[...]) + _dot(ag[0] + ag[1], w2r[...])
        xw_ref[...] = xw
        tp_ref[...] = xw * dv_ref[...]

    return pl.pallas_call(
        body, grid=(NHB,),
        in_specs=[pl.BlockSpec((RBLK, H), lambda i: (i, _Z())),
                  pl.BlockSpec((2, RBLK, H), lambda i: (_Z(), i, _Z())),
                  pl.BlockSpec((H, H), lambda i: (_Z(), _Z())),
                  pl.BlockSpec((H, H), lambda i: (_Z(), _Z())),
                  pl.BlockSpec((RBLK, H), lambda i: (i, _Z()))],
        out_specs=[pl.BlockSpec((RBLK, H), lambda i: (i, _Z()))] * 2,
        out_shape=[jax.ShapeDtypeStruct((NPAD, H), f32)] * 2,
    )(h_n, agg, w1, w2, dv)


def _node_post(h_n, xw_n, m, a1, dv, bn, won, bon):
    def body(hn_ref, xw_ref, m_ref, a1_ref, dv_ref, bnr, wor, bor, o_ref):
        mm = m_ref[...]
        pre = (xw_ref[...] * (a1_ref[...] + 1.0)
               + dv_ref[...] * (mm[0] + mm[1]) + bnr[...])
        nn = _dot(jnp.maximum(pre, 0.0), wor[...]) + bor[...]
        o_ref[...] = hn_ref[...] + nn

    return pl.pallas_call(
        body, grid=(NHB,),
        in_specs=[pl.BlockSpec((RBLK, H), lambda i: (i, _Z())),
                  pl.BlockSpec((RBLK, H), lambda i: (i, _Z())),
                  pl.BlockSpec((2, RBLK, H), lambda i: (_Z(), i, _Z())),
                  pl.BlockSpec((RBLK, H), lambda i: (i, _Z())),
                  pl.BlockSpec((RBLK, H), lambda i: (i, _Z())),
                  pl.BlockSpec((1, H), lambda i: (_Z(), _Z())),
                  pl.BlockSpec((H, H), lambda i: (_Z(), _Z())),
                  pl.BlockSpec((1, H), lambda i: (_Z(), _Z()))],
        out_specs=pl.BlockSpec((RBLK, H), lambda i: (i, _Z())),
        out_shape=jax.ShapeDtypeStruct((NPAD, H), f32),
    )(h_n, xw_n, m, a1, dv, bn.reshape(1, H), won, bon.reshape(1, H))


NBUF = 2


def _i32(v):
    return jnp.int32(v)


def _tile_ids():
    c = lax.axis_index("c")
    s = lax.axis_index("s")
    wid = c * _i32(16) + s
    return c, s, wid


def _zero_fill(buf2d):
    d = buf2d.shape[1]

    @pl.loop(_i32(0), _i32(CH))
    def _(r):
        for kk in range(0, d, 16):
            buf2d[r, pl.ds(_i32(kk), 16)] = jnp.zeros((16,), f32)


def _zero_stripe(zbuf, acc, s):
    for j in range(STRIPE // CH):
        pltpu.sync_copy(zbuf,
                        acc.at[pl.ds(s * _i32(STRIPE) + _i32(j * CH), CH)])


def _copy_out(acc, out_hbm, c, s):
    pltpu.sync_copy(
        acc.at[pl.ds(s * _i32(STRIPE), STRIPE)],
        out_hbm.at[pl.ds(c * _i32(NPAD) + s * _i32(STRIPE), STRIPE)])


def _load_idx(i2_hbm, i_t, wid):
    pltpu.sync_copy(i2_hbm.at[pl.ds(wid * _i32(NCH), NCH)], i_t)


def _sc_deg(dst2):
    @functools.partial(
        pl.kernel,
        out_type=jax.ShapeDtypeStruct((2 * NPAD, 16), f32),
        mesh=_mesh(),
        scratch_types=[pltpu.VMEM((NCH, CH), jnp.int32),
                       pltpu.VMEM((CH, 16), f32),
                       pltpu.VMEM((CH, 16), f32),
                       pltpu.VMEM_SHARED((NPAD, 16), f32),
                       pltpu.SemaphoreType.DMA],
    )
    def k(d_hbm, out_hbm, i2_t, ones_v, zbuf, acc, sem):
        c, s, wid = _tile_ids()
        _load_idx(d_hbm, i2_t, wid)
        _zero_fill(zbuf)
        _zero_stripe(zbuf, acc, s)

        @pl.loop(_i32(0), _i32(CH))
        def _(r):
            ones_v[r, pl.ds(_i32(0), 16)] = jnp.ones((16,), f32)

        plsc.subcore_barrier()

        @pl.loop(_i32(0), _i32(NCH // NBUF))
        def _(it):
            g0 = it * _i32(NBUF)
            cps = [pltpu.async_copy(ones_v, acc.at[i2_t.at[g0 + b]], sem,
                                    add=True)
                   for b in range(NBUF)]
            for cp in cps:
                cp.wait()

        plsc.subcore_barrier()
        _copy_out(acc, out_hbm, c, s)

    return k(dst2).reshape(2, NPAD, 16)


def _sc_gather2(A, B, src2, dst2):
    @functools.partial(
        pl.kernel,
        out_type=jax.ShapeDtypeStruct((EP, H), f32),
        mesh=_mesh(),
        scratch_types=[pltpu.VMEM((NCH, CH), jnp.int32),
                       pltpu.VMEM((NCH, CH), jnp.int32),
                       pltpu.VMEM((2, CH, H), f32),
                       pltpu.VMEM((2, CH, H), f32),
                       pltpu.SemaphoreType.DMA],
    )
    def k(a_hbm, b_hbm, s_hbm, d_hbm, g_hbm, i1_t, i2_t, bufa, bufb, sem):
        c, s, wid = _tile_ids()
        _load_idx(s_hbm, i1_t, wid)
        _load_idx(d_hbm, i2_t, wid)
        base0 = wid * _i32(PER_TILE)

        @pl.loop(_i32(0), _i32(NCH // 2))
        def _(it):
            g0 = it * _i32(2)
            cps = []
            for b in range(2):
                cps.append(pltpu.async_copy(
                    a_hbm.at[i1_t.at[g0 + b]], bufa.at[_i32(b)], sem))
                cps.append(pltpu.async_copy(
                    b_hbm.at[i2_t.at[g0 + b]], bufb.at[_i32(b)], sem))
            for cp in cps:
                cp.wait()
            for b in range(2):
                @pl.loop(_i32(0), _i32(CH))
                def _(r, _b=b):
                    for kk in range(0, H, 16):
                        sl = pl.ds(_i32(kk), 16)
                        bufa[_b, r, sl] = bufa[_b, r, sl] + bufb[_b, r, sl]
            scps = [pltpu.async_copy(
                bufa.at[_i32(b)],
                g_hbm.at[pl.ds(base0 + (g0 + b) * _i32(CH), CH)], sem)
                for b in range(2)]
            for cp in scps:
                cp.wait()

    return k(A, B, src2, dst2)


def _sc_msg(tab, src2, dst2):
    @functools.partial(
        pl.kernel,
        out_type=jax.ShapeDtypeStruct((2 * NPAD, H), f32),
        mesh=_mesh(),
        scratch_types=[pltpu.VMEM((NCH, CH), jnp.int32),
                       pltpu.VMEM((NCH, CH), jnp.int32),
                       pltpu.VMEM((NBUF, CH, H), f32),
                       pltpu.VMEM_SHARED((NPAD, H), f32),
                       pltpu.SemaphoreType.DMA,
                       pltpu.SemaphoreType.DMA],
    )
    def k(t_hbm, s_hbm, d_hbm, out_hbm, i1_t, i2_t, buf, acc, gsem, ssem):
        c, s, wid = _tile_ids()
        _load_idx(s_hbm, i1_t, wid)
        _load_idx(d_hbm, i2_t, wid)
        _zero_fill(buf.at[_i32(0)])
        _zero_stripe(buf.at[_i32(0)], acc, s)
        plsc.subcore_barrier()

        @pl.loop(_i32(0), _i32(NCH // NBUF))
        def _(it):
            g0 = it * _i32(NBUF)
            cps = [pltpu.async_copy(
                t_hbm.at[i1_t.at[g0 + b]], buf.at[_i32(b)], gsem)
                for b in range(NBUF)]
            for cp in cps:
                cp.wait()
            scps = [pltpu.async_copy(
                buf.at[_i32(b)], acc.at[i2_t.at[g0 + b]], ssem, add=True)
                for b in range(NBUF)]
            for cp in scps:
                cp.wait()

        plsc.subcore_barrier()
        _copy_out(acc, out_hbm, c, s)

    return k(tab, src2, dst2).reshape(2, NPAD, H)


def _sc_agg(V, dst2):
    @functools.partial(
        pl.kernel,
        out_type=jax.ShapeDtypeStruct((2 * NPAD, H), f32),
        mesh=_mesh(),
        scratch_types=[pltpu.VMEM((NCH, CH), jnp.int32),
                       pltpu.VMEM((NBUF, CH, H), f32),
                       pltpu.VMEM_SHARED((NPAD, H), f32),
                       pltpu.SemaphoreType.DMA,
                       pltpu.SemaphoreType.DMA],
    )
    def k(v_hbm, d_hbm, out_hbm, i2_t, buf, acc, gsem, ssem):
        c, s, wid = _tile_ids()
        _load_idx(d_hbm, i2_t, wid)
        _zero_fill(buf.at[_i32(0)])
        _zero_stripe(buf.at[_i32(0)], acc, s)
        plsc.subcore_barrier()
        base0 = wid * _i32(PER_TILE)

        @pl.loop(_i32(0), _i32(NCH // NBUF))
        def _(it):
            g0 = it * _i32(NBUF)
            cps = [pltpu.async_copy(
                v_hbm.at[pl.ds(base0 + (g0 + b) * _i32(CH), CH)],
                buf.at[_i32(b)], gsem)
                for b in range(NBUF)]
            for cp in cps:
                cp.wait()
            scps = [pltpu.async_copy(
                buf.at[_i32(b)], acc.at[i2_t.at[g0 + b]], ssem, add=True)
                for b in range(NBUF)]
            for cp in scps:
                cp.wait()

        plsc.subcore_barrier()
        _copy_out(acc, out_hbm, c, s)

    return k(V, dst2).reshape(2, NPAD, H)


def kernel(x, edge_index, edge_attr, params):
    src = edge_index[0].astype(jnp.int32)
    dst = edge_index[1].astype(jnp.int32)
    src2 = jnp.pad(src, (0, EP - E)).reshape(NTILES * NCH, CH)
    dst2g = jnp.pad(dst, (0, EP - E)).reshape(NTILES * NCH, CH)
    dst2s = (jnp.pad(dst, (0, EP - E), constant_values=N)
             .reshape(NTILES * NCH, CH))
    x_p = jnp.pad(x.astype(f32), ((0, NPAD - N), (0, 0)))
    ea_p = jnp.pad(edge_attr.astype(f32), ((0, EP - E), (0, 0)))

    h_n = _mlp(x_p, params["enc_node"], ln=True)
    h_e = _mlp(ea_p, params["enc_edge"], ln=True)
    cnt = _sc_deg(dst2s)
    dv, a1 = _deg_post(cnt)

    for blk in params["blocks"]:
        eb, nb = blk["eb"], blk["nb"]
        w1, w2, w3 = eb["Wc"][:H], eb["Wc"][H:2 * H], eb["Wc"][2 * H:]
        A, B = _ab(h_n, w1, w2)
        G = _sc_gather2(A, B, src2, dst2g)
        xw_h, tab = _head_pre(G, h_e, w3, dv)
        m = _sc_msg(tab, src2, dst2s)
        fx = _fix(xw_h, a1, dv, m)
        new_e, h_e = _edge_big(G, h_e, fx, w3, eb["bc"], eb["Wo"], eb["bo"])
        agg = _sc_agg(new_e, dst2s)
        wn1, wn2 = nb["Wc"][:H], nb["Wc"][H:]
        xw_n, tab_n = _node_pre(h_n, agg, wn1, wn2, dv)
        mn = _sc_msg(tab_n, src2, dst2s)
        h_n = _node_post(h_n, xw_n, mn, a1, dv, nb["bc"], nb["Wo"], nb["bo"])

    out = _mlp(h_n, params["dec"], ln=False)
    return out[:N, :2]


# speedup vs baseline: 6.1267x; 1.0304x over previous
import functools

import jax
import jax.numpy as jnp
from jax import lax
from jax.experimental import pallas as pl
from jax.experimental.pallas import tpu as pltpu
from jax.experimental.pallas import tpu_sc as plsc

N = 10000
E = 160000
H = 128
NPAD = 10240
EP = 163840
NTILES = 32
PER_TILE = EP // NTILES
CH = 128
NCH = PER_TILE // CH
STRIPE = NPAD // 16
RBLK = 2048
NHB = NPAD // RBLK

f32 = jnp.float32


@functools.cache
def _mesh():
    return plsc.VectorSubcoreMesh(core_axis_name="c", subcore_axis_name="s")


def _Z():
    return jnp.int32(0)


def _dot(a, b):
    return lax.dot_general(a, b, (((1,), (0,)), ((), ())),
                           preferred_element_type=f32)


def _silu(h):
    return h * (1.0 / (1.0 + jnp.exp(-h)))


def _mlp(xp, p, ln):
    n, din = xp.shape
    ws = [w.astype(f32) for w in p["Ws"]]
    dout = ws[3].shape[1]
    if dout != H:
        ws = ws[:3] + [jnp.pad(ws[3], ((0, 0), (0, H - dout)))]
    bs = jnp.stack([jnp.pad(b, (0, H - b.shape[0])) for b in p["bs"]])
    gb = (jnp.stack([p["ln_g"], p["ln_b"]]) if ln
          else jnp.zeros((2, H), f32))

    def body(x_ref, w0, w1, w2, w3, bs_ref, gb_ref, o_ref):
        h = x_ref[...]
        b = bs_ref[...]
        for i, w in enumerate((w0, w1, w2)):
            h = _silu(_dot(h, w[...]) + b[i])
        h = _dot(h, w3[...]) + b[3]
        if ln:
            g = gb_ref[...]
            mu = jnp.mean(h, axis=-1, keepdims=True)
            var = jnp.mean((h - mu) ** 2, axis=-1, keepdims=True)
            h = (h - mu) * lax.rsqrt(var + 1e-5) * g[0] + g[1]
        o_ref[...] = h

    wspec = [pl.BlockSpec((ws[i].shape[0], H), lambda i: (_Z(), _Z()))
             for i in range(4)]
    return pl.pallas_call(
        body, grid=(n // RBLK,),
        in_specs=[pl.BlockSpec((RBLK, din), lambda i: (i, _Z()))] + wspec + [
            pl.BlockSpec((4, H), lambda i: (_Z(), _Z())),
            pl.BlockSpec((2, H), lambda i: (_Z(), _Z()))],
        out_specs=pl.BlockSpec((RBLK, H), lambda i: (i, _Z())),
        out_shape=jax.ShapeDtypeStruct((n, H), f32),
    )(xp, *ws, bs, gb)


def _deg_post(cnt):
    def body(c_ref, dv_ref, a1_ref):
        cb = c_ref[...]
        deg = (cb[0] + cb[1])[:, 0:1] + 1.0
        row = (lax.broadcasted_iota(jnp.int32, (RBLK, 1), 0)
               + pl.program_id(0) * RBLK)
        mask = (row < N).astype(f32)
        dv = mask * lax.rsqrt(deg)
        a1 = mask * (1.0 / deg - 1.0)
        dv_ref[...] = jnp.broadcast_to(dv, (RBLK, H))
        a1_ref[...] = jnp.broadcast_to(a1, (RBLK, H))

    return pl.pallas_call(
        body, grid=(NHB,),
        in_specs=[pl.BlockSpec((2, RBLK, 16), lambda i: (_Z(), i, _Z()))],
        out_specs=[pl.BlockSpec((RBLK, H), lambda i: (i, _Z()))] * 2,
        out_shape=[jax.ShapeDtypeStruct((NPAD, H), f32)] * 2,
    )(cnt)


def _ab(h_n, w1, w2):
    def body(x_ref, w1r, w2r, a_ref, b_ref):
        xx = x_ref[...]
        a_ref[...] = _dot(xx, w1r[...])
        b_ref[...] = _dot(xx, w2r[...])

    return pl.pallas_call(
        body, grid=(NHB,),
        in_specs=[pl.BlockSpec((RBLK, H), lambda i: (i, _Z())),
                  pl.BlockSpec((H, H), lambda i: (_Z(), _Z())),
                  pl.BlockSpec((H, H), lambda i: (_Z(), _Z()))],
        out_specs=[pl.BlockSpec((RBLK, H), lambda i: (i, _Z()))] * 2,
        out_shape=[jax.ShapeDtypeStruct((NPAD, H), f32)] * 2,
    )(h_n, w1, w2)


def _head_pre(G, h_e, w3, dv):
    def body(g_ref, he_ref, w3r, dv_ref, xw_ref, tp_ref):
        xw = g_ref[...] + _dot(he_ref[...], w3r[...])
        xw_ref[...] = xw
        tp_ref[...] = xw * dv_ref[...]

    return pl.pallas_call(
        body, grid=(NHB,),
        in_specs=[pl.BlockSpec((RBLK, H), lambda i: (i, _Z())),
                  pl.BlockSpec((RBLK, H), lambda i: (i, _Z())),
                  pl.BlockSpec((H, H), lambda i: (_Z(), _Z())),
                  pl.BlockSpec((RBLK, H), lambda i: (i, _Z()))],
        out_specs=[pl.BlockSpec((RBLK, H), lambda i: (i, _Z()))] * 2,
        out_shape=[jax.ShapeDtypeStruct((NPAD, H), f32)] * 2,
    )(G, h_e, w3, dv)


def _fix(xw_h, a1, dv, m):
    def body(xw_ref, a1_ref, dv_ref, m_ref, o_ref):
        mm = m_ref[...]
        o_ref[...] = (xw_ref[...] * a1_ref[...]
                      + dv_ref[...] * (mm[0] + mm[1]))

    return pl.pallas_call(
        body, grid=(NHB,),
        in_specs=[pl.BlockSpec((RBLK, H), lambda i: (i, _Z())),
                  pl.BlockSpec((RBLK, H), lambda i: (i, _Z())),
                  pl.BlockSpec((RBLK, H), lambda i: (i, _Z())),
                  pl.BlockSpec((2, RBLK, H), lambda i: (_Z(), i, _Z()))],
        out_specs=pl.BlockSpec((RBLK, H), lambda i: (i, _Z())),
        out_shape=jax.ShapeDtypeStruct((NPAD, H), f32),
    )(xw_h, a1, dv, m)


ET = EP - NPAD
NTB = ET // RBLK


def _edge_big_head(G, he_h, fx, w3, bc, wo, bo):
    def body(g_ref, he_ref, fix_ref, w3r, bcr, wor, bor, ne_ref, hn_ref):
        he = he_ref[...]
        pre = (g_ref[...] + _dot(he, w3r[...]) + bcr[...] + fix_ref[...])
        ne = _dot(jnp.maximum(pre, 0.0), wor[...]) + bor[...]
        ne_ref[...] = ne
        hn_ref[...] = he + ne

    return pl.pallas_call(
        body, grid=(NHB,),
        in_specs=[pl.BlockSpec((RBLK, H), lambda i: (i, _Z())),
                  pl.BlockSpec((RBLK, H), lambda i: (i, _Z())),
                  pl.BlockSpec((RBLK, H), lambda i: (i, _Z())),
                  pl.BlockSpec((H, H), lambda i: (_Z(), _Z())),
                  pl.BlockSpec((1, H), lambda i: (_Z(), _Z())),
                  pl.BlockSpec((H, H), lambda i: (_Z(), _Z())),
                  pl.BlockSpec((1, H), lambda i: (_Z(), _Z()))],
        out_specs=[pl.BlockSpec((RBLK, H), lambda i: (i, _Z()))] * 2,
        out_shape=[jax.ShapeDtypeStruct((NPAD, H), f32)] * 2,
    )(G, he_h, fx, w3, bc.reshape(1, H), wo, bo.reshape(1, H))


def _edge_big_tail(G, he_t, w3, bc, wo, bo):
    def body(g_ref, he_ref, w3r, bcr, wor, bor, ne_ref, hn_ref):
        he = he_ref[...]
        pre = g_ref[...] + _dot(he, w3r[...]) + bcr[...]
        ne = _dot(jnp.maximum(pre, 0.0), wor[...]) + bor[...]
        ne_ref[...] = ne
        hn_ref[...] = he + ne

    return pl.pallas_call(
        body, grid=(NTB,),
        in_specs=[pl.BlockSpec((RBLK, H), lambda i: (i + jnp.int32(NHB),
                                                     _Z())),
                  pl.BlockSpec((RBLK, H), lambda i: (i, _Z())),
                  pl.BlockSpec((H, H), lambda i: (_Z(), _Z())),
                  pl.BlockSpec((1, H), lambda i: (_Z(), _Z())),
                  pl.BlockSpec((H, H), lambda i: (_Z(), _Z())),
                  pl.BlockSpec((1, H), lambda i: (_Z(), _Z()))],
        out_specs=[pl.BlockSpec((RBLK, H), lambda i: (i, _Z()))] * 2,
        out_shape=[jax.ShapeDtypeStruct((ET, H), f32)] * 2,
    )(G, he_t, w3, bc.reshape(1, H), wo, bo.reshape(1, H))


def _node_pre(h_n, agg, w1, w2, dv):
    def body(hn_ref, ag_ref, w1r, w2r, dv_ref, xw_ref, tp_ref):
        ag = ag_ref[...]
        xw = _dot(hn_ref[...], w1r[...]) + _dot(ag[0] + ag[1], w2r[...])
        xw_ref[...] = xw
        tp_ref[...] = xw * dv_ref[...]

    return pl.pallas_call(
        body, grid=(NHB,),
        in_specs=[pl.BlockSpec((RBLK, H), lambda i: (i, _Z())),
                  pl.BlockSpec((2, RBLK, H), lambda i: (_Z(), i, _Z())),
                  pl.BlockSpec((H, H), lambda i: (_Z(), _Z())),
                  pl.BlockSpec((H, H), lambda i: (_Z(), _Z())),
                  pl.BlockSpec((RBLK, H), lambda i: (i, _Z()))],
        out_specs=[pl.BlockSpec((RBLK, H), lambda i: (i, _Z()))] * 2,
        out_shape=[jax.ShapeDtypeStruct((NPAD, H), f32)] * 2,
    )(h_n, agg, w1, w2, dv)


def _node_post(h_n, xw_n, m, a1, dv, bn, won, bon):
    def body(hn_ref, xw_ref, m_ref, a1_ref, dv_ref, bnr, wor, bor, o_ref):
        mm = m_ref[...]
        pre = (xw_ref[...] * (a1_ref[...] + 1.0)
               + dv_ref[...] * (mm[0] + mm[1]) + bnr[...])
        nn = _dot(jnp.maximum(pre, 0.0), wor[...]) + bor[...]
        o_ref[...] = hn_ref[...] + nn

    return pl.pallas_call(
        body, grid=(NHB,),
        in_specs=[pl.BlockSpec((RBLK, H), lambda i: (i, _Z())),
                  pl.BlockSpec((RBLK, H), lambda i: (i, _Z())),
                  pl.BlockSpec((2, RBLK, H), lambda i: (_Z(), i, _Z())),
                  pl.BlockSpec((RBLK, H), lambda i: (i, _Z())),
                  pl.BlockSpec((RBLK, H), lambda i: (i, _Z())),
                  pl.BlockSpec((1, H), lambda i: (_Z(), _Z())),
                  pl.BlockSpec((H, H), lambda i: (_Z(), _Z())),
                  pl.BlockSpec((1, H), lambda i: (_Z(), _Z()))],
        out_specs=pl.BlockSpec((RBLK, H), lambda i: (i, _Z())),
        out_shape=jax.ShapeDtypeStruct((NPAD, H), f32),
    )(h_n, xw_n, m, a1, dv, bn.reshape(1, H), won, bon.reshape(1, H))


NBUF = 2


def _i32(v):
    return jnp.int32(v)


def _tile_ids():
    c = lax.axis_index("c")
    s = lax.axis_index("s")
    wid = c * _i32(16) + s
    return c, s, wid


def _zero_fill(buf2d):
    d = buf2d.shape[1]

    @pl.loop(_i32(0), _i32(CH))
    def _(r):
        for kk in range(0, d, 16):
            buf2d[r, pl.ds(_i32(kk), 16)] = jnp.zeros((16,), f32)


def _zero_stripe(zbuf, acc, s):
    for j in range(STRIPE // CH):
        pltpu.sync_copy(zbuf,
                        acc.at[pl.ds(s * _i32(STRIPE) + _i32(j * CH), CH)])


def _copy_out(acc, out_hbm, c, s):
    pltpu.sync_copy(
        acc.at[pl.ds(s * _i32(STRIPE), STRIPE)],
        out_hbm.at[pl.ds(c * _i32(NPAD) + s * _i32(STRIPE), STRIPE)])


def _load_idx(i2_hbm, i_t, wid):
    pltpu.sync_copy(i2_hbm.at[pl.ds(wid * _i32(NCH), NCH)], i_t)


def _sc_deg(dst2):
    @functools.partial(
        pl.kernel,
        out_type=jax.ShapeDtypeStruct((2 * NPAD, 16), f32),
        mesh=_mesh(),
        scratch_types=[pltpu.VMEM((NCH, CH), jnp.int32),
                       pltpu.VMEM((CH, 16), f32),
                       pltpu.VMEM((CH, 16), f32),
                       pltpu.VMEM_SHARED((NPAD, 16), f32),
                       pltpu.SemaphoreType.DMA],
    )
    def k(d_hbm, out_hbm, i2_t, ones_v, zbuf, acc, sem):
        c, s, wid = _tile_ids()
        _load_idx(d_hbm, i2_t, wid)
        _zero_fill(zbuf)
        _zero_stripe(zbuf, acc, s)

        @pl.loop(_i32(0), _i32(CH))
        def _(r):
            ones_v[r, pl.ds(_i32(0), 16)] = jnp.ones((16,), f32)

        plsc.subcore_barrier()

        @pl.loop(_i32(0), _i32(NCH // 8))
        def _(it):
            g0 = it * _i32(8)
            cps = [pltpu.async_copy(ones_v, acc.at[i2_t.at[g0 + b]], sem,
                                    add=True)
                   for b in range(8)]
            for cp in cps:
                cp.wait()

        plsc.subcore_barrier()
        _copy_out(acc, out_hbm, c, s)

    return k(dst2).reshape(2, NPAD, 16)


def _sc_gather2(A, B, src2, dst2):
    @functools.partial(
        pl.kernel,
        out_type=jax.ShapeDtypeStruct((EP, H), f32),
        mesh=_mesh(),
        scratch_types=[pltpu.VMEM((NCH, CH), jnp.int32),
                       pltpu.VMEM((NCH, CH), jnp.int32),
                       pltpu.VMEM((2, CH, H), f32),
                       pltpu.VMEM((2, CH, H), f32),
                       pltpu.SemaphoreType.DMA],
    )
    def k(a_hbm, b_hbm, s_hbm, d_hbm, g_hbm, i1_t, i2_t, bufa, bufb, sem):
        c, s, wid = _tile_ids()
        _load_idx(s_hbm, i1_t, wid)
        _load_idx(d_hbm, i2_t, wid)
        base0 = wid * _i32(PER_TILE)

        @pl.loop(_i32(0), _i32(NCH // 2))
        def _(it):
            g0 = it * _i32(2)
            cps = []
            for b in range(2):
                cps.append(pltpu.async_copy(
                    a_hbm.at[i1_t.at[g0 + b]], bufa.at[_i32(b)], sem))
                cps.append(pltpu.async_copy(
                    b_hbm.at[i2_t.at[g0 + b]], bufb.at[_i32(b)], sem))
            for cp in cps:
                cp.wait()
            for b in range(2):
                @pl.loop(_i32(0), _i32(CH))
                def _(r, _b=b):
                    for kk in range(0, H, 16):
                        sl = pl.ds(_i32(kk), 16)
                        bufa[_b, r, sl] = bufa[_b, r, sl] + bufb[_b, r, sl]
            scps = [pltpu.async_copy(
                bufa.at[_i32(b)],
                g_hbm.at[pl.ds(base0 + (g0 + b) * _i32(CH), CH)], sem)
                for b in range(2)]
            for cp in scps:
                cp.wait()

    return k(A, B, src2, dst2)


def _sc_msg(tab, src2, dst2):
    @functools.partial(
        pl.kernel,
        out_type=jax.ShapeDtypeStruct((2 * NPAD, H), f32),
        mesh=_mesh(),
        scratch_types=[pltpu.VMEM((NCH, CH), jnp.int32),
                       pltpu.VMEM((NCH, CH), jnp.int32),
                       pltpu.VMEM((NBUF, CH, H), f32),
                       pltpu.VMEM_SHARED((NPAD, H), f32),
                       pltpu.SemaphoreType.DMA,
                       pltpu.SemaphoreType.DMA],
    )
    def k(t_hbm, s_hbm, d_hbm, out_hbm, i1_t, i2_t, buf, acc, gsem, ssem):
        c, s, wid = _tile_ids()
        _load_idx(s_hbm, i1_t, wid)
        _load_idx(d_hbm, i2_t, wid)
        _zero_fill(buf.at[_i32(0)])
        _zero_stripe(buf.at[_i32(0)], acc, s)
        plsc.subcore_barrier()

        @pl.loop(_i32(0), _i32(NCH // NBUF))
        def _(it):
            g0 = it * _i32(NBUF)
            cps = [pltpu.async_copy(
                t_hbm.at[i1_t.at[g0 + b]], buf.at[_i32(b)], gsem)
                for b in range(NBUF)]
            for cp in cps:
                cp.wait()
            scps = [pltpu.async_copy(
                buf.at[_i32(b)], acc.at[i2_t.at[g0 + b]], ssem, add=True)
                for b in range(NBUF)]
            for cp in scps:
                cp.wait()

        plsc.subcore_barrier()
        _copy_out(acc, out_hbm, c, s)

    return k(tab, src2, dst2).reshape(2, NPAD, H)


def _sc_agg(Vh, Vt, dst2):
    @functools.partial(
        pl.kernel,
        out_type=jax.ShapeDtypeStruct((2 * NPAD, H), f32),
        mesh=_mesh(),
        scratch_types=[pltpu.VMEM((NCH, CH), jnp.int32),
                       pltpu.VMEM((NBUF, CH, H), f32),
                       pltpu.VMEM_SHARED((NPAD, H), f32),
                       pltpu.SemaphoreType.DMA,
                       pltpu.SemaphoreType.DMA],
    )
    def k(vh_hbm, vt_hbm, d_hbm, out_hbm, i2_t, buf, acc, gsem, ssem):
        c, s, wid = _tile_ids()
        _load_idx(d_hbm, i2_t, wid)
        _zero_fill(buf.at[_i32(0)])
        _zero_stripe(buf.at[_i32(0)], acc, s)
        plsc.subcore_barrier()

        def run(v_hbm, base0):
            @pl.loop(_i32(0), _i32(NCH // NBUF))
            def _(it):
                g0 = it * _i32(NBUF)
                cps = [pltpu.async_copy(
                    v_hbm.at[pl.ds(base0 + (g0 + b) * _i32(CH), CH)],
                    buf.at[_i32(b)], gsem)
                    for b in range(NBUF)]
                for cp in cps:
                    cp.wait()
                scps = [pltpu.async_copy(
                    buf.at[_i32(b)], acc.at[i2_t.at[g0 + b]], ssem, add=True)
                    for b in range(NBUF)]
                for cp in scps:
                    cp.wait()

        @pl.when(wid < 2)
        def _():
            run(vh_hbm, wid * _i32(PER_TILE))

        @pl.when(wid >= 2)
        def _():
            run(vt_hbm, wid * _i32(PER_TILE) - _i32(NPAD))

        plsc.subcore_barrier()
        _copy_out(acc, out_hbm, c, s)

    return k(Vh, Vt, dst2).reshape(2, NPAD, H)


def kernel(x, edge_index, edge_attr, params):
    src = edge_index[0].astype(jnp.int32)
    dst = edge_index[1].astype(jnp.int32)
    src2 = jnp.pad(src, (0, EP - E)).reshape(NTILES * NCH, CH)
    dst2g = jnp.pad(dst, (0, EP - E)).reshape(NTILES * NCH, CH)
    dst2s = (jnp.pad(dst, (0, EP - E), constant_values=N)
             .reshape(NTILES * NCH, CH))
    x_p = jnp.pad(x.astype(f32), ((0, NPAD - N), (0, 0)))
    ea_p = jnp.pad(edge_attr.astype(f32), ((0, EP - E), (0, 0)))

    h_n = _mlp(x_p, params["enc_node"], ln=True)
    he_h = _mlp(ea_p[:NPAD], params["enc_edge"], ln=True)
    he_t = _mlp(ea_p[NPAD:], params["enc_edge"], ln=True)
    cnt = _sc_deg(dst2s)
    dv, a1 = _deg_post(cnt)

    for blk in params["blocks"]:
        eb, nb = blk["eb"], blk["nb"]
        w1, w2, w3 = eb["Wc"][:H], eb["Wc"][H:2 * H], eb["Wc"][2 * H:]
        A, B = _ab(h_n, w1, w2)
        G = _sc_gather2(A, B, src2, dst2g)
        xw_h, tab = _head_pre(G, he_h, w3, dv)
        m = _sc_msg(tab, src2, dst2s)
        ne_t, he_t = _edge_big_tail(G, he_t, w3, eb["bc"], eb["Wo"],
                                    eb["bo"])
        fx = _fix(xw_h, a1, dv, m)
        ne_h, he_h = _edge_big_head(G, he_h, fx, w3, eb["bc"], eb["Wo"],
                                    eb["bo"])
        agg = _sc_agg(ne_h, ne_t, dst2s)
        wn1, wn2 = nb["Wc"][:H], nb["Wc"][H:]
        xw_n, tab_n = _node_pre(h_n, agg, wn1, wn2, dv)
        mn = _sc_msg(tab_n, src2, dst2s)
        h_n = _node_post(h_n, xw_n, mn, a1, dv, nb["bc"], nb["Wo"], nb["bo"])

    out = _mlp(h_n, params["dec"], ln=False)
    return out[:N, :2]


# speedup vs baseline: 6.5742x; 1.0730x over previous
import functools

import jax
import jax.numpy as jnp
from jax import lax
from jax.experimental import pallas as pl
from jax.experimental.pallas import tpu as pltpu
from jax.experimental.pallas import tpu_sc as plsc

N = 10000
E = 160000
H = 128
NPAD = 10240
EP = 163840
NTILES = 32
PER_TILE = EP // NTILES
CH = 128
NCH = PER_TILE // CH
STRIPE = NPAD // 16
RBLK = 2048
NHB = NPAD // RBLK

f32 = jnp.float32


@functools.cache
def _mesh():
    return plsc.VectorSubcoreMesh(core_axis_name="c", subcore_axis_name="s")


def _Z():
    return jnp.int32(0)


def _dot(a, b):
    return lax.dot_general(a, b, (((1,), (0,)), ((), ())),
                           preferred_element_type=f32)


def _silu(h):
    return h * (1.0 / (1.0 + jnp.exp(-h)))


def _mlp(xp, p, ln):
    n, din = xp.shape
    ws = [w.astype(f32) for w in p["Ws"]]
    dout = ws[3].shape[1]
    if dout != H:
        ws = ws[:3] + [jnp.pad(ws[3], ((0, 0), (0, H - dout)))]
    bs = jnp.stack([jnp.pad(b, (0, H - b.shape[0])) for b in p["bs"]])
    gb = (jnp.stack([p["ln_g"], p["ln_b"]]) if ln
          else jnp.zeros((2, H), f32))

    def body(x_ref, w0, w1, w2, w3, bs_ref, gb_ref, o_ref):
        h = x_ref[...]
        b = bs_ref[...]
        for i, w in enumerate((w0, w1, w2)):
            h = _silu(_dot(h, w[...]) + b[i])
        h = _dot(h, w3[...]) + b[3]
        if ln:
            g = gb_ref[...]
            mu = jnp.mean(h, axis=-1, keepdims=True)
            var = jnp.mean((h - mu) ** 2, axis=-1, keepdims=True)
            h = (h - mu) * lax.rsqrt(var + 1e-5) * g[0] + g[1]
        o_ref[...] = h

    wspec = [pl.BlockSpec((ws[i].shape[0], H), lambda i: (_Z(), _Z()))
             for i in range(4)]
    return pl.pallas_call(
        body, grid=(n // RBLK,),
        in_specs=[pl.BlockSpec((RBLK, din), lambda i: (i, _Z()))] + wspec + [
            pl.BlockSpec((4, H), lambda i: (_Z(), _Z())),
            pl.BlockSpec((2, H), lambda i: (_Z(), _Z()))],
        out_specs=pl.BlockSpec((RBLK, H), lambda i: (i, _Z())),
        out_shape=jax.ShapeDtypeStruct((n, H), f32),
    )(xp, *ws, bs, gb)


def _deg_post(cnt):
    def body(c_ref, dv_ref, a1_ref):
        cb = c_ref[...]
        deg = (cb[0] + cb[1])[:, 0:1] + 1.0
        row = (lax.broadcasted_iota(jnp.int32, (RBLK, 1), 0)
               + pl.program_id(0) * RBLK)
        mask = (row < N).astype(f32)
        dv = mask * lax.rsqrt(deg)
        a1 = mask * (1.0 / deg - 1.0)
        dv_ref[...] = jnp.broadcast_to(dv, (RBLK, H))
        a1_ref[...] = jnp.broadcast_to(a1, (RBLK, H))

    return pl.pallas_call(
        body, grid=(NHB,),
        in_specs=[pl.BlockSpec((2, RBLK, 16), lambda i: (_Z(), i, _Z()))],
        out_specs=[pl.BlockSpec((RBLK, H), lambda i: (i, _Z()))] * 2,
        out_shape=[jax.ShapeDtypeStruct((NPAD, H), f32)] * 2,
    )(cnt)


def _ab(h_n, w1, w2):
    def body(x_ref, w1r, w2r, a_ref, b_ref):
        xx = x_ref[...]
        a_ref[...] = _dot(xx, w1r[...])
        b_ref[...] = _dot(xx, w2r[...])

    return pl.pallas_call(
        body, grid=(NHB,),
        in_specs=[pl.BlockSpec((RBLK, H), lambda i: (i, _Z())),
                  pl.BlockSpec((H, H), lambda i: (_Z(), _Z())),
                  pl.BlockSpec((H, H), lambda i: (_Z(), _Z()))],
        out_specs=[pl.BlockSpec((RBLK, H), lambda i: (i, _Z()))] * 2,
        out_shape=[jax.ShapeDtypeStruct((NPAD, H), f32)] * 2,
    )(h_n, w1, w2)


def _head_pre(G, h_e, w3, dv):
    def body(g_ref, he_ref, w3r, dv_ref, xw_ref, tp_ref):
        xw = g_ref[...] + _dot(he_ref[...], w3r[...])
        xw_ref[...] = xw
        tp_ref[...] = xw * dv_ref[...]

    return pl.pallas_call(
        body, grid=(NHB,),
        in_specs=[pl.BlockSpec((RBLK, H), lambda i: (i, _Z())),
                  pl.BlockSpec((RBLK, H), lambda i: (i, _Z())),
                  pl.BlockSpec((H, H), lambda i: (_Z(), _Z())),
                  pl.BlockSpec((RBLK, H), lambda i: (i, _Z()))],
        out_specs=[pl.BlockSpec((RBLK, H), lambda i: (i, _Z()))] * 2,
        out_shape=[jax.ShapeDtypeStruct((NPAD, H), f32)] * 2,
    )(G, h_e, w3, dv)


def _fix(xw_h, a1, dv, m):
    def body(xw_ref, a1_ref, dv_ref, m_ref, o_ref):
        mm = m_ref[...]
        o_ref[...] = (xw_ref[...] * a1_ref[...]
                      + dv_ref[...] * (mm[0] + mm[1]))

    return pl.pallas_call(
        body, grid=(NHB,),
        in_specs=[pl.BlockSpec((RBLK, H), lambda i: (i, _Z())),
                  pl.BlockSpec((RBLK, H), lambda i: (i, _Z())),
                  pl.BlockSpec((RBLK, H), lambda i: (i, _Z())),
                  pl.BlockSpec((2, RBLK, H), lambda i: (_Z(), i, _Z()))],
        out_specs=pl.BlockSpec((RBLK, H), lambda i: (i, _Z())),
        out_shape=jax.ShapeDtypeStruct((NPAD, H), f32),
    )(xw_h, a1, dv, m)


ET = EP - NPAD
NTB = ET // RBLK


def _edge_big_head(G, he_h, fx, w3, bc, wo, bo):
    def body(g_ref, he_ref, fix_ref, w3r, bcr, wor, bor, ne_ref, hn_ref):
        he = he_ref[...]
        pre = (g_ref[...] + _dot(he, w3r[...]) + bcr[...] + fix_ref[...])
        ne = _dot(jnp.maximum(pre, 0.0), wor[...]) + bor[...]
        ne_ref[...] = ne
        hn_ref[...] = he + ne

    return pl.pallas_call(
        body, grid=(NHB,),
        in_specs=[pl.BlockSpec((RBLK, H), lambda i: (i, _Z())),
                  pl.BlockSpec((RBLK, H), lambda i: (i, _Z())),
                  pl.BlockSpec((RBLK, H), lambda i: (i, _Z())),
                  pl.BlockSpec((H, H), lambda i: (_Z(), _Z())),
                  pl.BlockSpec((1, H), lambda i: (_Z(), _Z())),
                  pl.BlockSpec((H, H), lambda i: (_Z(), _Z())),
                  pl.BlockSpec((1, H), lambda i: (_Z(), _Z()))],
        out_specs=[pl.BlockSpec((RBLK, H), lambda i: (i, _Z()))] * 2,
        out_shape=[jax.ShapeDtypeStruct((NPAD, H), f32)] * 2,
    )(G, he_h, fx, w3, bc.reshape(1, H), wo, bo.reshape(1, H))


def _edge_big_tail(G, he_t, w3, bc, wo, bo):
    def body(g_ref, he_ref, w3r, bcr, wor, bor, ne_ref, hn_ref):
        he = he_ref[...]
        pre = g_ref[...] + _dot(he, w3r[...]) + bcr[...]
        ne = _dot(jnp.maximum(pre, 0.0), wor[...]) + bor[...]
        ne_ref[...] = ne
        hn_ref[...] = he + ne

    return pl.pallas_call(
        body, grid=(NTB,),
        in_specs=[pl.BlockSpec((RBLK, H), lambda i: (i + jnp.int32(NHB),
                                                     _Z())),
                  pl.BlockSpec((RBLK, H), lambda i: (i, _Z())),
                  pl.BlockSpec((H, H), lambda i: (_Z(), _Z())),
                  pl.BlockSpec((1, H), lambda i: (_Z(), _Z())),
                  pl.BlockSpec((H, H), lambda i: (_Z(), _Z())),
                  pl.BlockSpec((1, H), lambda i: (_Z(), _Z()))],
        out_specs=[pl.BlockSpec((RBLK, H), lambda i: (i, _Z()))] * 2,
        out_shape=[jax.ShapeDtypeStruct((ET, H), f32)] * 2,
    )(G, he_t, w3, bc.reshape(1, H), wo, bo.reshape(1, H))


def _node_pre(h_n, agg, w1, w2, dv):
    def body(hn_ref, ag_ref, w1r, w2r, dv_ref, xw_ref, tp_ref):
        ag = ag_ref[...]
        xw = _dot(hn_ref[...], w1r[...]) + _dot(ag[0] + ag[1], w2r[...])
        xw_ref[...] = xw
        tp_ref[...] = xw * dv_ref[...]

    return pl.pallas_call(
        body, grid=(NHB,),
        in_specs=[pl.BlockSpec((RBLK, H), lambda i: (i, _Z())),
                  pl.BlockSpec((2, RBLK, H), lambda i: (_Z(), i, _Z())),
                  pl.BlockSpec((H, H), lambda i: (_Z(), _Z())),
                  pl.BlockSpec((H, H), lambda i: (_Z(), _Z())),
                  pl.BlockSpec((RBLK, H), lambda i: (i, _Z()))],
        out_specs=[pl.BlockSpec((RBLK, H), lambda i: (i, _Z()))] * 2,
        out_shape=[jax.ShapeDtypeStruct((NPAD, H), f32)] * 2,
    )(h_n, agg, w1, w2, dv)


def _node_post(h_n, xw_n, m, a1, dv, bn, won, bon):
    def body(hn_ref, xw_ref, m_ref, a1_ref, dv_ref, bnr, wor, bor, o_ref):
        mm = m_ref[...]
        pre = (xw_ref[...] * (a1_ref[...] + 1.0)
               + dv_ref[...] * (mm[0] + mm[1]) + bnr[...])
        nn = _dot(jnp.maximum(pre, 0.0), wor[...]) + bor[...]
        o_ref[...] = hn_ref[...] + nn

    return pl.pallas_call(
        body, grid=(NHB,),
        in_specs=[pl.BlockSpec((RBLK, H), lambda i: (i, _Z())),
                  pl.BlockSpec((RBLK, H), lambda i: (i, _Z())),
                  pl.BlockSpec((2, RBLK, H), lambda i: (_Z(), i, _Z())),
                  pl.BlockSpec((RBLK, H), lambda i: (i, _Z())),
                  pl.BlockSpec((RBLK, H), lambda i: (i, _Z())),
                  pl.BlockSpec((1, H), lambda i: (_Z(), _Z())),
                  pl.BlockSpec((H, H), lambda i: (_Z(), _Z())),
                  pl.BlockSpec((1, H), lambda i: (_Z(), _Z()))],
        out_specs=pl.BlockSpec((RBLK, H), lambda i: (i, _Z())),
        out_shape=jax.ShapeDtypeStruct((NPAD, H), f32),
    )(h_n, xw_n, m, a1, dv, bn.reshape(1, H), won, bon.reshape(1, H))


NBUF = 2


def _i32(v):
    return jnp.int32(v)


def _tile_ids():
    c = lax.axis_index("c")
    s = lax.axis_index("s")
    wid = c * _i32(16) + s
    return c, s, wid


def _zero_fill(buf2d):
    d = buf2d.shape[1]

    @pl.loop(_i32(0), _i32(CH))
    def _(r):
        for kk in range(0, d, 16):
            buf2d[r, pl.ds(_i32(kk), 16)] = jnp.zeros((16,), f32)


def _zero_stripe(zbuf, acc, s):
    for j in range(STRIPE // CH):
        pltpu.sync_copy(zbuf,
                        acc.at[pl.ds(s * _i32(STRIPE) + _i32(j * CH), CH)])


def _copy_out(acc, out_hbm, c, s):
    pltpu.sync_copy(
        acc.at[pl.ds(s * _i32(STRIPE), STRIPE)],
        out_hbm.at[pl.ds(c * _i32(NPAD) + s * _i32(STRIPE), STRIPE)])


def _load_idx(i2_hbm, i_t, wid):
    pltpu.sync_copy(i2_hbm.at[pl.ds(wid * _i32(NCH), NCH)], i_t)


def _sc_deg(dst2):
    @functools.partial(
        pl.kernel,
        out_type=jax.ShapeDtypeStruct((2 * NPAD, 16), f32),
        mesh=_mesh(),
        scratch_types=[pltpu.VMEM((NCH, CH), jnp.int32),
                       pltpu.VMEM((CH, 16), f32),
                       pltpu.VMEM((CH, 16), f32),
                       pltpu.VMEM_SHARED((NPAD, 16), f32),
                       pltpu.SemaphoreType.DMA],
    )
    def k(d_hbm, out_hbm, i2_t, ones_v, zbuf, acc, sem):
        c, s, wid = _tile_ids()
        _load_idx(d_hbm, i2_t, wid)
        _zero_fill(zbuf)
        _zero_stripe(zbuf, acc, s)

        @pl.loop(_i32(0), _i32(CH))
        def _(r):
            ones_v[r, pl.ds(_i32(0), 16)] = jnp.ones((16,), f32)

        plsc.subcore_barrier()

        @pl.loop(_i32(0), _i32(NCH // 8))
        def _(it):
            g0 = it * _i32(8)
            cps = [pltpu.async_copy(ones_v, acc.at[i2_t.at[g0 + b]], sem,
                                    add=True)
                   for b in range(8)]
            for cp in cps:
                cp.wait()

        plsc.subcore_barrier()
        _copy_out(acc, out_hbm, c, s)

    return k(dst2).reshape(2, NPAD, 16)


def _sc_gather2(A, B, src2, dst2):
    @functools.partial(
        pl.kernel,
        out_type=jax.ShapeDtypeStruct((EP, H), f32),
        mesh=_mesh(),
        scratch_types=[pltpu.VMEM((NCH, CH), jnp.int32),
                       pltpu.VMEM((NCH, CH), jnp.int32),
                       pltpu.VMEM((2, CH, H), f32),
                       pltpu.VMEM((2, CH, H), f32),
                       pltpu.VMEM((2, CH, H), f32),
                       pltpu.SemaphoreType.DMA,
                       pltpu.SemaphoreType.DMA,
                       pltpu.SemaphoreType.DMA,
                       pltpu.SemaphoreType.DMA],
    )
    def k(a_hbm, b_hbm, s_hbm, d_hbm, g_hbm, i1_t, i2_t, bufa, bufb, bufc,
          ga0, ga1, st0, st1):
        c, s, wid = _tile_ids()
        _load_idx(s_hbm, i1_t, wid)
        _load_idx(d_hbm, i2_t, wid)
        base0 = wid * _i32(PER_TILE)
        gsems = (ga0, ga1)
        ssems = (st0, st1)

        def issue_gather(g, b):
            pltpu.async_copy(a_hbm.at[i1_t.at[g]], bufa.at[_i32(b)],
                             gsems[b])
            pltpu.async_copy(b_hbm.at[i2_t.at[g]], bufb.at[_i32(b)],
                             gsems[b])

        for b in range(2):
            issue_gather(_i32(b), b)

        @pl.loop(_i32(0), _i32(NCH // 2))
        def _(it):
            for b in range(2):
                g = it * _i32(2) + _i32(b)
                off = base0 + g * _i32(CH)
                pltpu.make_async_copy(a_hbm.at[i1_t.at[g]],
                                      bufa.at[_i32(b)], gsems[b]).wait()
                pltpu.make_async_copy(b_hbm.at[i2_t.at[g]],
                                      bufb.at[_i32(b)], gsems[b]).wait()

                @pl.when(it > _i32(0))
                def _(_b=b):
                    pltpu.make_async_copy(
                        bufc.at[_i32(_b)], g_hbm.at[pl.ds(base0, CH)],
                        ssems[_b]).wait()

                @pl.loop(_i32(0), _i32(CH))
                def _(r, _b=b):
                    for kk in range(0, H, 16):
                        sl = pl.ds(_i32(kk), 16)
                        bufc[_b, r, sl] = bufa[_b, r, sl] + bufb[_b, r, sl]

                @pl.when(it < _i32(NCH // 2 - 1))
                def _(_b=b, _g=g):
                    issue_gather(_g + _i32(2), _b)

                pltpu.async_copy(bufc.at[_i32(b)],
                                 g_hbm.at[pl.ds(off, CH)], ssems[b])

        for b in range(2):
            pltpu.make_async_copy(bufc.at[_i32(b)],
                                  g_hbm.at[pl.ds(base0, CH)],
                                  ssems[b]).wait()

    return k(A, B, src2, dst2)


def _sc_msg(tab, src2, dst2):
    @functools.partial(
        pl.kernel,
        out_type=jax.ShapeDtypeStruct((2 * NPAD, H), f32),
        mesh=_mesh(),
        scratch_types=[pltpu.VMEM((NCH, CH), jnp.int32),
                       pltpu.VMEM((NCH, CH), jnp.int32),
                       pltpu.VMEM((NBUF, CH, H), f32),
                       pltpu.VMEM_SHARED((NPAD, H), f32),
                       pltpu.SemaphoreType.DMA,
                       pltpu.SemaphoreType.DMA],
    )
    def k(t_hbm, s_hbm, d_hbm, out_hbm, i1_t, i2_t, buf, acc, gsem, ssem):
        c, s, wid = _tile_ids()
        _load_idx(s_hbm, i1_t, wid)
        _load_idx(d_hbm, i2_t, wid)
        _zero_fill(buf.at[_i32(0)])
        _zero_stripe(buf.at[_i32(0)], acc, s)
        plsc.subcore_barrier()

        @pl.loop(_i32(0), _i32(NCH // 2))
        def _(it):
            g0 = it * _i32(2)
            cps = [pltpu.async_copy(
                t_hbm.at[i1_t.at[g0 + b]], buf.at[_i32(b)], gsem)
                for b in range(2)]
            for cp in cps:
                cp.wait()
            scps = [pltpu.async_copy(
                buf.at[_i32(b)], acc.at[i2_t.at[g0 + b]], ssem, add=True)
                for b in range(2)]
            for cp in scps:
                cp.wait()

        plsc.subcore_barrier()
        _copy_out(acc, out_hbm, c, s)

    return k(tab, src2, dst2).reshape(2, NPAD, H)


def _sc_agg(Vh, Vt, dst2):
    @functools.partial(
        pl.kernel,
        out_type=jax.ShapeDtypeStruct((2 * NPAD, H), f32),
        mesh=_mesh(),
        scratch_types=[pltpu.VMEM((NCH, CH), jnp.int32),
                       pltpu.VMEM((NBUF, CH, H), f32),
                       pltpu.VMEM_SHARED((NPAD, H), f32),
                       pltpu.SemaphoreType.DMA,
                       pltpu.SemaphoreType.DMA],
    )
    def k(vh_hbm, vt_hbm, d_hbm, out_hbm, i2_t, buf, acc, gsem, ssem):
        c, s, wid = _tile_ids()
        _load_idx(d_hbm, i2_t, wid)
        _zero_fill(buf.at[_i32(0)])
        _zero_stripe(buf.at[_i32(0)], acc, s)
        plsc.subcore_barrier()

        def run(v_hbm, base0):
            @pl.loop(_i32(0), _i32(NCH // 2))
            def _(it):
                g0 = it * _i32(2)
                cps = [pltpu.async_copy(
                    v_hbm.at[pl.ds(base0 + (g0 + b) * _i32(CH), CH)],
                    buf.at[_i32(b)], gsem)
                    for b in range(2)]
                for cp in cps:
                    cp.wait()
                scps = [pltpu.async_copy(
                    buf.at[_i32(b)], acc.at[i2_t.at[g0 + b]], ssem,
                    add=True)
                    for b in range(2)]
                for cp in scps:
                    cp.wait()

        @pl.when(wid < 2)
        def _():
            run(vh_hbm, wid * _i32(PER_TILE))

        @pl.when(wid >= 2)
        def _():
            run(vt_hbm, wid * _i32(PER_TILE) - _i32(NPAD))

        plsc.subcore_barrier()
        _copy_out(acc, out_hbm, c, s)

    return k(Vh, Vt, dst2).reshape(2, NPAD, H)


def kernel(x, edge_index, edge_attr, params):
    src = edge_index[0].astype(jnp.int32)
    dst = edge_index[1].astype(jnp.int32)
    src2 = jnp.pad(src, (0, EP - E)).reshape(NTILES * NCH, CH)
    dst2g = jnp.pad(dst, (0, EP - E)).reshape(NTILES * NCH, CH)
    dst2s = (jnp.pad(dst, (0, EP - E), constant_values=N)
             .reshape(NTILES * NCH, CH))
    x_p = jnp.pad(x.astype(f32), ((0, NPAD - N), (0, 0)))
    ea_p = jnp.pad(edge_attr.astype(f32), ((0, EP - E), (0, 0)))

    h_n = _mlp(x_p, params["enc_node"], ln=True)
    he_h = _mlp(ea_p[:NPAD], params["enc_edge"], ln=True)
    he_t = _mlp(ea_p[NPAD:], params["enc_edge"], ln=True)
    cnt = _sc_deg(dst2s)
    dv, a1 = _deg_post(cnt)

    for blk in params["blocks"]:
        eb, nb = blk["eb"], blk["nb"]
        w1, w2, w3 = eb["Wc"][:H], eb["Wc"][H:2 * H], eb["Wc"][2 * H:]
        A, B = _ab(h_n, w1, w2)
        G = _sc_gather2(A, B, src2, dst2g)
        xw_h, tab = _head_pre(G, he_h, w3, dv)
        m = _sc_msg(tab, src2, dst2s)
        ne_t, he_t = _edge_big_tail(G, he_t, w3, eb["bc"], eb["Wo"],
                                    eb["bo"])
        fx = _fix(xw_h, a1, dv, m)
        ne_h, he_h = _edge_big_head(G, he_h, fx, w3, eb["bc"], eb["Wo"],
                                    eb["bo"])
        agg = _sc_agg(ne_h, ne_t, dst2s)
        wn1, wn2 = nb["Wc"][:H], nb["Wc"][H:]
        xw_n, tab_n = _node_pre(h_n, agg, wn1, wn2, dv)
        mn = _sc_msg(tab_n, src2, dst2s)
        h_n = _node_post(h_n, xw_n, mn, a1, dv, nb["bc"], nb["Wo"], nb["bo"])

    out = _mlp(h_n, params["dec"], ln=False)
    return out[:N, :2]


# speedup vs baseline: 6.9738x; 1.0608x over previous
import functools

import jax
import jax.numpy as jnp
from jax import lax
from jax.experimental import pallas as pl
from jax.experimental.pallas import tpu as pltpu
from jax.experimental.pallas import tpu_sc as plsc

N = 10000
E = 160000
H = 128
NPAD = 10240
EP = 163840
NTILES = 32
PER_TILE = EP // NTILES
CH = 128
NCH = PER_TILE // CH
STRIPE = NPAD // 16
RBLK = 2048
NHB = NPAD // RBLK

f32 = jnp.float32


@functools.cache
def _mesh():
    return plsc.VectorSubcoreMesh(core_axis_name="c", subcore_axis_name="s")


def _Z():
    return jnp.int32(0)


def _dot(a, b):
    return lax.dot_general(a, b, (((1,), (0,)), ((), ())),
                           preferred_element_type=f32)


def _silu(h):
    return h * (1.0 / (1.0 + jnp.exp(-h)))


def _mlp(xp, p, ln):
    n, din = xp.shape
    ws = [w.astype(f32) for w in p["Ws"]]
    dout = ws[3].shape[1]
    if dout != H:
        ws = ws[:3] + [jnp.pad(ws[3], ((0, 0), (0, H - dout)))]
    bs = jnp.stack([jnp.pad(b, (0, H - b.shape[0])) for b in p["bs"]])
    gb = (jnp.stack([p["ln_g"], p["ln_b"]]) if ln
          else jnp.zeros((2, H), f32))

    def body(x_ref, w0, w1, w2, w3, bs_ref, gb_ref, o_ref):
        h = x_ref[...]
        b = bs_ref[...]
        for i, w in enumerate((w0, w1, w2)):
            h = _silu(_dot(h, w[...]) + b[i])
        h = _dot(h, w3[...]) + b[3]
        if ln:
            g = gb_ref[...]
            mu = jnp.mean(h, axis=-1, keepdims=True)
            var = jnp.mean((h - mu) ** 2, axis=-1, keepdims=True)
            h = (h - mu) * lax.rsqrt(var + 1e-5) * g[0] + g[1]
        o_ref[...] = h

    wspec = [pl.BlockSpec((ws[i].shape[0], H), lambda i: (_Z(), _Z()))
             for i in range(4)]
    return pl.pallas_call(
        body, grid=(n // RBLK,),
        in_specs=[pl.BlockSpec((RBLK, din), lambda i: (i, _Z()))] + wspec + [
            pl.BlockSpec((4, H), lambda i: (_Z(), _Z())),
            pl.BlockSpec((2, H), lambda i: (_Z(), _Z()))],
        out_specs=pl.BlockSpec((RBLK, H), lambda i: (i, _Z())),
        out_shape=jax.ShapeDtypeStruct((n, H), f32),
    )(xp, *ws, bs, gb)


def _deg_post(cnt):
    def body(c_ref, dv_ref, a1_ref):
        cb = c_ref[...]
        deg = (cb[0] + cb[1])[:, 0:1] + 1.0
        row = (lax.broadcasted_iota(jnp.int32, (RBLK, 1), 0)
               + pl.program_id(0) * RBLK)
        mask = (row < N).astype(f32)
        dv = mask * lax.rsqrt(deg)
        a1 = mask * (1.0 / deg - 1.0)
        dv_ref[...] = jnp.broadcast_to(dv, (RBLK, H))
        a1_ref[...] = jnp.broadcast_to(a1, (RBLK, H))

    return pl.pallas_call(
        body, grid=(NHB,),
        in_specs=[pl.BlockSpec((2, RBLK, 16), lambda i: (_Z(), i, _Z()))],
        out_specs=[pl.BlockSpec((RBLK, H), lambda i: (i, _Z()))] * 2,
        out_shape=[jax.ShapeDtypeStruct((NPAD, H), f32)] * 2,
    )(cnt)


def _ab(h_n, w1, w2):
    def body(x_ref, w1r, w2r, a_ref, b_ref):
        xx = x_ref[...]
        a_ref[...] = _dot(xx, w1r[...])
        b_ref[...] = _dot(xx, w2r[...])

    return pl.pallas_call(
        body, grid=(NHB,),
        in_specs=[pl.BlockSpec((RBLK, H), lambda i: (i, _Z())),
                  pl.BlockSpec((H, H), lambda i: (_Z(), _Z())),
                  pl.BlockSpec((H, H), lambda i: (_Z(), _Z()))],
        out_specs=[pl.BlockSpec((RBLK, H), lambda i: (i, _Z()))] * 2,
        out_shape=[jax.ShapeDtypeStruct((NPAD, H), f32)] * 2,
    )(h_n, w1, w2)


def _head_pre(G, h_e, w3, dv):
    def body(g_ref, he_ref, w3r, dv_ref, xw_ref, tp_ref):
        xw = g_ref[...] + _dot(he_ref[...], w3r[...])
        xw_ref[...] = xw
        tp_ref[...] = xw * dv_ref[...]

    return pl.pallas_call(
        body, grid=(NHB,),
        in_specs=[pl.BlockSpec((RBLK, H), lambda i: (i, _Z())),
                  pl.BlockSpec((RBLK, H), lambda i: (i, _Z())),
                  pl.BlockSpec((H, H), lambda i: (_Z(), _Z())),
                  pl.BlockSpec((RBLK, H), lambda i: (i, _Z()))],
        out_specs=[pl.BlockSpec((RBLK, H), lambda i: (i, _Z()))] * 2,
        out_shape=[jax.ShapeDtypeStruct((NPAD, H), f32)] * 2,
    )(G, h_e, w3, dv)


def _fix(xw_h, a1, dv, m):
    def body(xw_ref, a1_ref, dv_ref, m_ref, o_ref):
        mm = m_ref[...]
        o_ref[...] = (xw_ref[...] * a1_ref[...]
                      + dv_ref[...] * (mm[0] + mm[1]))

    return pl.pallas_call(
        body, grid=(NHB,),
        in_specs=[pl.BlockSpec((RBLK, H), lambda i: (i, _Z())),
                  pl.BlockSpec((RBLK, H), lambda i: (i, _Z())),
                  pl.BlockSpec((RBLK, H), lambda i: (i, _Z())),
                  pl.BlockSpec((2, RBLK, H), lambda i: (_Z(), i, _Z()))],
        out_specs=pl.BlockSpec((RBLK, H), lambda i: (i, _Z())),
        out_shape=jax.ShapeDtypeStruct((NPAD, H), f32),
    )(xw_h, a1, dv, m)


ET = EP - NPAD
NTB = ET // RBLK


def _edge_big_head(G, he_h, fx, w3, bc, wo, bo):
    def body(g_ref, he_ref, fix_ref, w3r, bcr, wor, bor, ne_ref, hn_ref):
        he = he_ref[...]
        pre = (g_ref[...] + _dot(he, w3r[...]) + bcr[...] + fix_ref[...])
        ne = _dot(jnp.maximum(pre, 0.0), wor[...]) + bor[...]
        ne_ref[...] = ne
        hn_ref[...] = he + ne

    return pl.pallas_call(
        body, grid=(NHB,),
        in_specs=[pl.BlockSpec((RBLK, H), lambda i: (i, _Z())),
                  pl.BlockSpec((RBLK, H), lambda i: (i, _Z())),
                  pl.BlockSpec((RBLK, H), lambda i: (i, _Z())),
                  pl.BlockSpec((H, H), lambda i: (_Z(), _Z())),
                  pl.BlockSpec((1, H), lambda i: (_Z(), _Z())),
                  pl.BlockSpec((H, H), lambda i: (_Z(), _Z())),
                  pl.BlockSpec((1, H), lambda i: (_Z(), _Z()))],
        out_specs=[pl.BlockSpec((RBLK, H), lambda i: (i, _Z()))] * 2,
        out_shape=[jax.ShapeDtypeStruct((NPAD, H), f32)] * 2,
    )(G, he_h, fx, w3, bc.reshape(1, H), wo, bo.reshape(1, H))


def _edge_big_tail(G, he_t, w3, bc, wo, bo):
    def body(g_ref, he_ref, w3r, bcr, wor, bor, ne_ref, hn_ref):
        he = he_ref[...]
        pre = g_ref[...] + _dot(he, w3r[...]) + bcr[...]
        ne = _dot(jnp.maximum(pre, 0.0), wor[...]) + bor[...]
        ne_ref[...] = ne
        hn_ref[...] = he + ne

    return pl.pallas_call(
        body, grid=(NTB,),
        in_specs=[pl.BlockSpec((RBLK, H), lambda i: (i + jnp.int32(NHB),
                                                     _Z())),
                  pl.BlockSpec((RBLK, H), lambda i: (i, _Z())),
                  pl.BlockSpec((H, H), lambda i: (_Z(), _Z())),
                  pl.BlockSpec((1, H), lambda i: (_Z(), _Z())),
                  pl.BlockSpec((H, H), lambda i: (_Z(), _Z())),
                  pl.BlockSpec((1, H), lambda i: (_Z(), _Z()))],
        out_specs=[pl.BlockSpec((RBLK, H), lambda i: (i, _Z()))] * 2,
        out_shape=[jax.ShapeDtypeStruct((ET, H), f32)] * 2,
    )(G, he_t, w3, bc.reshape(1, H), wo, bo.reshape(1, H))


def _node_pre(h_n, agg, w1, w2, dv):
    def body(hn_ref, ag_ref, w1r, w2r, dv_ref, xw_ref, tp_ref):
        ag = ag_ref[...]
        xw = _dot(hn_ref[...], w1r[...]) + _dot(ag[0] + ag[1], w2r[...])
        xw_ref[...] = xw
        tp_ref[...] = xw * dv_ref[...]

    return pl.pallas_call(
        body, grid=(NHB,),
        in_specs=[pl.BlockSpec((RBLK, H), lambda i: (i, _Z())),
                  pl.BlockSpec((2, RBLK, H), lambda i: (_Z(), i, _Z())),
                  pl.BlockSpec((H, H), lambda i: (_Z(), _Z())),
                  pl.BlockSpec((H, H), lambda i: (_Z(), _Z())),
                  pl.BlockSpec((RBLK, H), lambda i: (i, _Z()))],
        out_specs=[pl.BlockSpec((RBLK, H), lambda i: (i, _Z()))] * 2,
        out_shape=[jax.ShapeDtypeStruct((NPAD, H), f32)] * 2,
    )(h_n, agg, w1, w2, dv)


def _node_post(h_n, xw_n, m, a1, dv, bn, won, bon):
    def body(hn_ref, xw_ref, m_ref, a1_ref, dv_ref, bnr, wor, bor, o_ref):
        mm = m_ref[...]
        pre = (xw_ref[...] * (a1_ref[...] + 1.0)
               + dv_ref[...] * (mm[0] + mm[1]) + bnr[...])
        nn = _dot(jnp.maximum(pre, 0.0), wor[...]) + bor[...]
        o_ref[...] = hn_ref[...] + nn

    return pl.pallas_call(
        body, grid=(NHB,),
        in_specs=[pl.BlockSpec((RBLK, H), lambda i: (i, _Z())),
                  pl.BlockSpec((RBLK, H), lambda i: (i, _Z())),
                  pl.BlockSpec((2, RBLK, H), lambda i: (_Z(), i, _Z())),
                  pl.BlockSpec((RBLK, H), lambda i: (i, _Z())),
                  pl.BlockSpec((RBLK, H), lambda i: (i, _Z())),
                  pl.BlockSpec((1, H), lambda i: (_Z(), _Z())),
                  pl.BlockSpec((H, H), lambda i: (_Z(), _Z())),
                  pl.BlockSpec((1, H), lambda i: (_Z(), _Z()))],
        out_specs=pl.BlockSpec((RBLK, H), lambda i: (i, _Z())),
        out_shape=jax.ShapeDtypeStruct((NPAD, H), f32),
    )(h_n, xw_n, m, a1, dv, bn.reshape(1, H), won, bon.reshape(1, H))


NBUF = 2


def _i32(v):
    return jnp.int32(v)


def _tile_ids():
    c = lax.axis_index("c")
    s = lax.axis_index("s")
    wid = c * _i32(16) + s
    return c, s, wid


def _zero_fill(buf2d):
    d = buf2d.shape[1]

    @pl.loop(_i32(0), _i32(CH))
    def _(r):
        for kk in range(0, d, 16):
            buf2d[r, pl.ds(_i32(kk), 16)] = jnp.zeros((16,), f32)


def _zero_stripe(zbuf, acc, s):
    for j in range(STRIPE // CH):
        pltpu.sync_copy(zbuf,
                        acc.at[pl.ds(s * _i32(STRIPE) + _i32(j * CH), CH)])


def _copy_out(acc, out_hbm, c, s):
    pltpu.sync_copy(
        acc.at[pl.ds(s * _i32(STRIPE), STRIPE)],
        out_hbm.at[pl.ds(c * _i32(NPAD) + s * _i32(STRIPE), STRIPE)])


def _load_idx(i2_hbm, i_t, wid):
    pltpu.sync_copy(i2_hbm.at[pl.ds(wid * _i32(NCH), NCH)], i_t)


def _sc_deg(dst2):
    @functools.partial(
        pl.kernel,
        out_type=jax.ShapeDtypeStruct((2 * NPAD, 16), f32),
        mesh=_mesh(),
        scratch_types=[pltpu.VMEM((NCH, CH), jnp.int32),
                       pltpu.VMEM((CH, 16), f32),
                       pltpu.VMEM((CH, 16), f32),
                       pltpu.VMEM_SHARED((NPAD, 16), f32),
                       pltpu.SemaphoreType.DMA],
    )
    def k(d_hbm, out_hbm, i2_t, ones_v, zbuf, acc, sem):
        c, s, wid = _tile_ids()
        _load_idx(d_hbm, i2_t, wid)
        _zero_fill(zbuf)
        _zero_stripe(zbuf, acc, s)

        @pl.loop(_i32(0), _i32(CH))
        def _(r):
            ones_v[r, pl.ds(_i32(0), 16)] = jnp.ones((16,), f32)

        plsc.subcore_barrier()

        @pl.loop(_i32(0), _i32(NCH // 8))
        def _(it):
            g0 = it * _i32(8)
            cps = [pltpu.async_copy(ones_v, acc.at[i2_t.at[g0 + b]], sem,
                                    add=True)
                   for b in range(8)]
            for cp in cps:
                cp.wait()

        plsc.subcore_barrier()
        _copy_out(acc, out_hbm, c, s)

    return k(dst2).reshape(2, NPAD, 16)


def _sc_gather2(A, B, src2, dst2):
    @functools.partial(
        pl.kernel,
        out_type=jax.ShapeDtypeStruct((EP, H), f32),
        mesh=_mesh(),
        scratch_types=[pltpu.VMEM((NCH, CH), jnp.int32),
                       pltpu.VMEM((NCH, CH), jnp.int32),
                       pltpu.VMEM((2, CH, H), f32),
                       pltpu.VMEM((2, CH, H), f32),
                       pltpu.VMEM((2, CH, H), f32),
                       pltpu.SemaphoreType.DMA,
                       pltpu.SemaphoreType.DMA,
                       pltpu.SemaphoreType.DMA,
                       pltpu.SemaphoreType.DMA],
    )
    def k(a_hbm, b_hbm, s_hbm, d_hbm, g_hbm, i1_t, i2_t, bufa, bufb, bufc,
          ga0, ga1, st0, st1):
        c, s, wid = _tile_ids()
        _load_idx(s_hbm, i1_t, wid)
        _load_idx(d_hbm, i2_t, wid)
        base0 = wid * _i32(PER_TILE)
        gsems = (ga0, ga1)
        ssems = (st0, st1)

        def issue_gather(g, b):
            pltpu.async_copy(a_hbm.at[i1_t.at[g]], bufa.at[_i32(b)],
                             gsems[b])
            pltpu.async_copy(b_hbm.at[i2_t.at[g]], bufb.at[_i32(b)],
                             gsems[b])

        for b in range(2):
            issue_gather(_i32(b), b)

        @pl.loop(_i32(0), _i32(NCH // 2))
        def _(it):
            for b in range(2):
                g = it * _i32(2) + _i32(b)
                off = base0 + g * _i32(CH)
                pltpu.make_async_copy(a_hbm.at[i1_t.at[g]],
                                      bufa.at[_i32(b)], gsems[b]).wait()
                pltpu.make_async_copy(b_hbm.at[i2_t.at[g]],
                                      bufb.at[_i32(b)], gsems[b]).wait()

                @pl.when(it > _i32(0))
                def _(_b=b):
                    pltpu.make_async_copy(
                        bufc.at[_i32(_b)], g_hbm.at[pl.ds(base0, CH)],
                        ssems[_b]).wait()

                @pl.loop(_i32(0), _i32(CH))
                def _(r, _b=b):
                    for kk in range(0, H, 16):
                        sl = pl.ds(_i32(kk), 16)
                        bufc[_b, r, sl] = bufa[_b, r, sl] + bufb[_b, r, sl]

                @pl.when(it < _i32(NCH // 2 - 1))
                def _(_b=b, _g=g):
                    issue_gather(_g + _i32(2), _b)

                pltpu.async_copy(bufc.at[_i32(b)],
                                 g_hbm.at[pl.ds(off, CH)], ssems[b])

        for b in range(2):
            pltpu.make_async_copy(bufc.at[_i32(b)],
                                  g_hbm.at[pl.ds(base0, CH)],
                                  ssems[b]).wait()

    return k(A, B, src2, dst2)


def _sc_msg(tab, src2, dst2):
    @functools.partial(
        pl.kernel,
        out_type=jax.ShapeDtypeStruct((2 * NPAD, H), f32),
        mesh=_mesh(),
        scratch_types=[pltpu.VMEM((NCH, CH), jnp.int32),
                       pltpu.VMEM((NCH, CH), jnp.int32),
                       pltpu.VMEM((NBUF, CH, H), f32),
                       pltpu.VMEM_SHARED((NPAD, H), f32),
                       pltpu.SemaphoreType.DMA,
                       pltpu.SemaphoreType.DMA,
                       pltpu.SemaphoreType.DMA,
                       pltpu.SemaphoreType.DMA],
    )
    def k(t_hbm, s_hbm, d_hbm, out_hbm, i1_t, i2_t, buf, acc, gsem, ssem,
          gsem2, ssem2):
        c, s, wid = _tile_ids()
        _load_idx(s_hbm, i1_t, wid)
        _load_idx(d_hbm, i2_t, wid)
        _zero_fill(buf.at[_i32(0)])
        _zero_stripe(buf.at[_i32(0)], acc, s)
        plsc.subcore_barrier()

        gsems = (gsem, gsem2)
        ssems = (ssem, ssem2)
        for b in range(2):
            pltpu.async_copy(t_hbm.at[i1_t.at[_i32(b)]], buf.at[_i32(b)],
                             gsems[b])

        @pl.loop(_i32(0), _i32(NCH // 2))
        def _(it):
            for b in range(2):
                g = it * _i32(2) + _i32(b)
                pltpu.make_async_copy(t_hbm.at[i1_t.at[g]],
                                      buf.at[_i32(b)], gsems[b]).wait()
                cp = pltpu.async_copy(
                    buf.at[_i32(b)], acc.at[i2_t.at[g]], ssems[b],
                    add=True)
                cp.wait()

                @pl.when(it < _i32(NCH // 2 - 1))
                def _(_b=b, _g=g):
                    pltpu.async_copy(t_hbm.at[i1_t.at[_g + _i32(2)]],
                                     buf.at[_i32(_b)], gsems[_b])

        plsc.subcore_barrier()
        _copy_out(acc, out_hbm, c, s)

    return k(tab, src2, dst2).reshape(2, NPAD, H)


def _sc_agg(Vh, Vt, dst2):
    @functools.partial(
        pl.kernel,
        out_type=jax.ShapeDtypeStruct((2 * NPAD, H), f32),
        mesh=_mesh(),
        scratch_types=[pltpu.VMEM((NCH, CH), jnp.int32),
                       pltpu.VMEM((NBUF, CH, H), f32),
                       pltpu.VMEM_SHARED((NPAD, H), f32),
                       pltpu.SemaphoreType.DMA,
                       pltpu.SemaphoreType.DMA,
                       pltpu.SemaphoreType.DMA,
                       pltpu.SemaphoreType.DMA],
    )
    def k(vh_hbm, vt_hbm, d_hbm, out_hbm, i2_t, buf, acc, gsem, ssem,
          gsem2, ssem2):
        c, s, wid = _tile_ids()
        _load_idx(d_hbm, i2_t, wid)
        _zero_fill(buf.at[_i32(0)])
        _zero_stripe(buf.at[_i32(0)], acc, s)
        plsc.subcore_barrier()

        gsems = (gsem, gsem2)
        ssems = (ssem, ssem2)

        def run(v_hbm, base0):
            for b in range(2):
                pltpu.async_copy(
                    v_hbm.at[pl.ds(base0 + _i32(b * CH), CH)],
                    buf.at[_i32(b)], gsems[b])

            @pl.loop(_i32(0), _i32(NCH // 2))
            def _(it):
                for b in range(2):
                    g = it * _i32(2) + _i32(b)
                    pltpu.make_async_copy(
                        v_hbm.at[pl.ds(base0 + g * _i32(CH), CH)],
                        buf.at[_i32(b)], gsems[b]).wait()
                    cp = pltpu.async_copy(
                        buf.at[_i32(b)], acc.at[i2_t.at[g]], ssems[b],
                        add=True)
                    cp.wait()

                    @pl.when(it < _i32(NCH // 2 - 1))
                    def _(_b=b, _g=g):
                        pltpu.async_copy(
                            v_hbm.at[pl.ds(base0 + (_g + _i32(2)) * _i32(CH),
                                           CH)],
                            buf.at[_i32(_b)], gsems[_b])

        @pl.when(wid < 2)
        def _():
            run(vh_hbm, wid * _i32(PER_TILE))

        @pl.when(wid >= 2)
        def _():
            run(vt_hbm, wid * _i32(PER_TILE) - _i32(NPAD))

        plsc.subcore_barrier()
        _copy_out(acc, out_hbm, c, s)

    return k(Vh, Vt, dst2).reshape(2, NPAD, H)


def kernel(x, edge_index, edge_attr, params):
    src = edge_index[0].astype(jnp.int32)
    dst = edge_index[1].astype(jnp.int32)
    src2 = jnp.pad(src, (0, EP - E)).reshape(NTILES * NCH, CH)
    dst2g = jnp.pad(dst, (0, EP - E)).reshape(NTILES * NCH, CH)
    dst2s = (jnp.pad(dst, (0, EP - E), constant_values=N)
             .reshape(NTILES * NCH, CH))
    x_p = jnp.pad(x.astype(f32), ((0, NPAD - N), (0, 0)))
    ea_p = jnp.pad(edge_attr.astype(f32), ((0, EP - E), (0, 0)))

    h_n = _mlp(x_p, params["enc_node"], ln=True)
    he_h = _mlp(ea_p[:NPAD], params["enc_edge"], ln=True)
    he_t = _mlp(ea_p[NPAD:], params["enc_edge"], ln=True)
    cnt = _sc_deg(dst2s)
    dv, a1 = _deg_post(cnt)

    for blk in params["blocks"]:
        eb, nb = blk["eb"], blk["nb"]
        w1, w2, w3 = eb["Wc"][:H], eb["Wc"][H:2 * H], eb["Wc"][2 * H:]
        A, B = _ab(h_n, w1, w2)
        G = _sc_gather2(A, B, src2, dst2g)
        xw_h, tab = _head_pre(G, he_h, w3, dv)
        m = _sc_msg(tab, src2, dst2s)
        ne_t, he_t = _edge_big_tail(G, he_t, w3, eb["bc"], eb["Wo"],
                                    eb["bo"])
        fx = _fix(xw_h, a1, dv, m)
        ne_h, he_h = _edge_big_head(G, he_h, fx, w3, eb["bc"], eb["Wo"],
                                    eb["bo"])
        agg = _sc_agg(ne_h, ne_t, dst2s)
        wn1, wn2 = nb["Wc"][:H], nb["Wc"][H:]
        xw_n, tab_n = _node_pre(h_n, agg, wn1, wn2, dv)
        mn = _sc_msg(tab_n, src2, dst2s)
        h_n = _node_post(h_n, xw_n, mn, a1, dv, nb["bc"], nb["Wo"], nb["bo"])

    out = _mlp(h_n, params["dec"], ln=False)
    return out[:N, :2]
